# Initial kernel scaffold; baseline (speedup 1.0000x reference)
import functools

import jax
import jax.numpy as jnp
from jax import lax
from jax.experimental import pallas as pl
from jax.experimental.pallas import tpu as pltpu
from jax.experimental.pallas import tpu_sc as plsc

NC = 2
NS = 16
NW = NC * NS
CHUNK = 128
NUM_GRAPHS = 64


def _mesh():
    return plsc.VectorSubcoreMesh(core_axis_name="c", subcore_axis_name="s")


def _fill(ref, rows, d, value):
    @pl.loop(0, rows)
    def _(r):
        @pl.loop(0, d, step=16)
        def _(c):
            ref[r, pl.ds(c, 16)] = jnp.full((16,), value, jnp.float32)


def _sc_degree(dst_p, n_acc, d, n_chunks):
    @functools.partial(
        pl.kernel,
        out_type=jax.ShapeDtypeStruct((NC, n_acc, d), jnp.float32),
        mesh=_mesh(),
        scratch_types=[
            pltpu.VMEM_SHARED((n_acc, d), jnp.float32),
            pltpu.VMEM((2, CHUNK), jnp.int32),
            pltpu.VMEM((CHUNK, d), jnp.float32),
            pltpu.VMEM((64, d), jnp.float32),
        ],
    )
    def k(dst_hbm, out_hbm, acc, dstb, onesb, zb):
        cid = lax.axis_index("c")
        sid = lax.axis_index("s")
        wid = cid * NS + sid
        _fill(zb, 64, d, 0.0)
        _fill(onesb, CHUNK, d, 1.0)
        rps = n_acc // NS

        @pl.loop(0, rps // 64)
        def _(i):
            pltpu.sync_copy(zb, acc.at[pl.ds(sid * rps + i * 64, 64)])

        plsc.subcore_barrier()
        ew = n_chunks * CHUNK

        @pl.loop(0, n_chunks)
        def _(ci):
            off = wid * ew + ci * CHUNK
            pltpu.sync_copy(dst_hbm.at[pl.ds(off, CHUNK)], dstb.at[0])
            pltpu.sync_copy(onesb, acc.at[dstb.at[0]], add=True)

        plsc.subcore_barrier()

        @pl.loop(0, rps // 64)
        def _(i):
            r0 = sid * rps + i * 64
            pltpu.sync_copy(acc.at[pl.ds(r0, 64)], out_hbm.at[cid, pl.ds(r0, 64)])

    return k(dst_p)


def _sc_edge_pass(g, src_p, dst_p, n_acc, n_chunks):
    d = g.shape[1]

    @functools.partial(
        pl.kernel,
        out_type=jax.ShapeDtypeStruct((NC, n_acc, d), jnp.float32),
        mesh=_mesh(),
        scratch_types=[
            pltpu.VMEM_SHARED((n_acc, d), jnp.float32),
            pltpu.VMEM((2, CHUNK), jnp.int32),
            pltpu.VMEM((2, CHUNK), jnp.int32),
            pltpu.VMEM((2, CHUNK, d), jnp.float32),
            pltpu.VMEM((64, d), jnp.float32),
        ],
    )
    def k(g_hbm, src_hbm, dst_hbm, out_hbm, acc, srcb, dstb, rows, zb):
        cid = lax.axis_index("c")
        sid = lax.axis_index("s")
        wid = cid * NS + sid
        _fill(zb, 64, d, 0.0)
        rps = n_acc // NS

        @pl.loop(0, rps // 64)
        def _(i):
            pltpu.sync_copy(zb, acc.at[pl.ds(sid * rps + i * 64, 64)])

        plsc.subcore_barrier()
        ew = n_chunks * CHUNK

        @pl.loop(0, n_chunks)
        def _(ci):
            off = wid * ew + ci * CHUNK
            pltpu.sync_copy(src_hbm.at[pl.ds(off, CHUNK)], srcb.at[0])
            pltpu.sync_copy(dst_hbm.at[pl.ds(off, CHUNK)], dstb.at[0])
            pltpu.sync_copy(g_hbm.at[srcb.at[0]], rows.at[0])
            pltpu.sync_copy(rows.at[0], acc.at[dstb.at[0]], add=True)

        plsc.subcore_barrier()

        @pl.loop(0, rps // 64)
        def _(i):
            r0 = sid * rps + i * 64
            pltpu.sync_copy(acc.at[pl.ds(r0, 64)], out_hbm.at[cid, pl.ds(r0, 64)])

    return k(g, src_p, dst_p)


_DOT = dict(preferred_element_type=jnp.float32, precision=lax.Precision.HIGHEST)


def _tc_matmul(x, w):
    n, d = x.shape
    rb = 2000 if n % 2000 == 0 else n

    def body(x_ref, w_ref, o_ref):
        o_ref[...] = jnp.dot(x_ref[...], w_ref[...], **_DOT)

    return pl.pallas_call(
        body,
        grid=(n // rb,),
        in_specs=[
            pl.BlockSpec((rb, d), lambda i: (i, 0)),
            pl.BlockSpec((d, w.shape[1]), lambda i: (0, 0)),
        ],
        out_specs=pl.BlockSpec((rb, w.shape[1]), lambda i: (i, 0)),
        out_shape=jax.ShapeDtypeStruct((n, w.shape[1]), jnp.float32),
    )(x, w)


def _tc_scale(h1, degs):
    n, d = h1.shape
    rb = 2000 if n % 2000 == 0 else n

    def body(h_ref, degs_ref, g_ref, dinv_ref):
        deg = degs_ref[0] + degs_ref[1] + 1.0
        dinv = lax.rsqrt(jnp.maximum(deg, 1.0))
        dinv_ref[...] = dinv
        g_ref[...] = h_ref[...] * dinv

    return pl.pallas_call(
        body,
        grid=(n // rb,),
        in_specs=[
            pl.BlockSpec((rb, d), lambda i: (i, 0)),
            pl.BlockSpec((2, rb, d), lambda i: (0, i, 0)),
        ],
        out_specs=[
            pl.BlockSpec((rb, d), lambda i: (i, 0)),
            pl.BlockSpec((rb, d), lambda i: (i, 0)),
        ],
        out_shape=[
            jax.ShapeDtypeStruct((n, d), jnp.float32),
            jax.ShapeDtypeStruct((n, d), jnp.float32),
        ],
    )(h1, degs)


def _tc_mid(acc, g1, dinvb, b1, w2):
    n, d = g1.shape
    rb = 2000 if n % 2000 == 0 else n

    def body(acc_ref, g_ref, dinv_ref, b_ref, w_ref, o_ref):
        s = acc_ref[0] + acc_ref[1] + g_ref[...]
        h = jnp.maximum(dinv_ref[...] * s + b_ref[...], 0.0)
        o_ref[...] = jnp.dot(h, w_ref[...], **_DOT) * dinv_ref[...]

    return pl.pallas_call(
        body,
        grid=(n // rb,),
        in_specs=[
            pl.BlockSpec((2, rb, d), lambda i: (0, i, 0)),
            pl.BlockSpec((rb, d), lambda i: (i, 0)),
            pl.BlockSpec((rb, d), lambda i: (i, 0)),
            pl.BlockSpec((1, d), lambda i: (0, 0)),
            pl.BlockSpec((d, d), lambda i: (0, 0)),
        ],
        out_specs=pl.BlockSpec((rb, d), lambda i: (i, 0)),
        out_shape=jax.ShapeDtypeStruct((n, d), jnp.float32),
    )(acc, g1, dinvb, b1, w2)


def _tc_final(acc, g2, dinvb, b2, batch2d, wc, bc):
    n, d = g2.shape
    dout = wc.shape[1]
    rb = 2000 if n % 2000 == 0 else n
    nblk = n // rb

    def body(acc_ref, g_ref, dinv_ref, b_ref, batch_ref, wc_ref, bc_ref,
             o_ref, s_ref, c_ref):
        i = pl.program_id(0)

        @pl.when(i == 0)
        def _():
            s_ref[...] = jnp.zeros_like(s_ref)
            c_ref[...] = jnp.zeros_like(c_ref)

        s = acc_ref[0] + acc_ref[1] + g_ref[...]
        h = jnp.maximum(dinv_ref[...] * s + b_ref[...], 0.0)
        seg = lax.broadcasted_iota(jnp.int32, (NUM_GRAPHS, rb), 0)
        maskt = (seg == batch_ref[...]).astype(jnp.float32)
        s_ref[...] += jnp.dot(maskt, h, **_DOT)
        c_ref[...] = c_ref[...] + jnp.sum(maskt, axis=1, keepdims=True)

        @pl.when(i == nblk - 1)
        def _():
            pooled = s_ref[...] / jnp.maximum(c_ref[...], 1.0)
            o_ref[...] = jnp.dot(pooled, wc_ref[...], **_DOT) + bc_ref[...]

    return pl.pallas_call(
        body,
        grid=(nblk,),
        in_specs=[
            pl.BlockSpec((2, rb, d), lambda i: (0, i, 0)),
            pl.BlockSpec((rb, d), lambda i: (i, 0)),
            pl.BlockSpec((rb, d), lambda i: (i, 0)),
            pl.BlockSpec((1, d), lambda i: (0, 0)),
            pl.BlockSpec((1, rb), lambda i: (0, i)),
            pl.BlockSpec((d, dout), lambda i: (0, 0)),
            pl.BlockSpec((1, dout), lambda i: (0, 0)),
        ],
        out_specs=pl.BlockSpec((NUM_GRAPHS, dout), lambda i: (0, 0)),
        out_shape=jax.ShapeDtypeStruct((NUM_GRAPHS, dout), jnp.float32),
        scratch_shapes=[
            pltpu.VMEM((NUM_GRAPHS, d), jnp.float32),
            pltpu.VMEM((NUM_GRAPHS, d), jnp.float32),
        ],
    )(acc, g2, dinvb, b2, batch2d, wc, bc)


def kernel(x, edge_index, batch, W1, b1, W2, b2, Wc, bc):
    n, d = x.shape
    src = edge_index[0].astype(jnp.int32)
    dst = edge_index[1].astype(jnp.int32)
    e = src.shape[0]

    n_acc = -(-n // 1024) * 1024
    n_chunks = -(-e // (NW * CHUNK))
    e_pad = NW * n_chunks * CHUNK
    npad = e_pad - e
    pad_idx = jnp.arange(npad, dtype=jnp.int32)
    src_p = jnp.concatenate([src, pad_idx % n])
    dst_p = jnp.concatenate([dst, n + pad_idx % (n_acc - n)])

    degs = _sc_degree(dst_p, n_acc, d, n_chunks)
    h1 = _tc_matmul(x, W1)
    g1, dinvb = _tc_scale(h1, degs[:, :n, :])
    acc1 = _sc_edge_pass(g1, src_p, dst_p, n_acc, n_chunks)
    g2 = _tc_mid(acc1[:, :n, :], g1, dinvb, b1.reshape(1, -1), W2)
    acc2 = _sc_edge_pass(g2, src_p, dst_p, n_acc, n_chunks)
    return _tc_final(acc2[:, :n, :], g2, dinvb, b2.reshape(1, -1),
                     batch.astype(jnp.int32).reshape(1, -1), Wc,
                     bc.reshape(1, -1))


# speedup vs baseline: 15.3915x; 15.3915x over previous
import functools

import jax
import jax.numpy as jnp
from jax import lax
from jax.experimental import pallas as pl
from jax.experimental.pallas import tpu as pltpu
from jax.experimental.pallas import tpu_sc as plsc

NC = 2
NS = 16
NW = NC * NS
CHUNK = 128
NUM_GRAPHS = 64


def _mesh():
    return plsc.VectorSubcoreMesh(core_axis_name="c", subcore_axis_name="s")


def _fill(ref, rows, d, value):
    @pl.loop(0, rows)
    def _(r):
        @pl.loop(0, d, step=16)
        def _(c):
            ref[r, pl.ds(c, 16)] = jnp.full((16,), value, jnp.float32)


def _sc_degree(dst_p, n_acc, d, n_chunks):
    @functools.partial(
        pl.kernel,
        out_type=jax.ShapeDtypeStruct((NC, n_acc, d), jnp.float32),
        mesh=_mesh(),
        scratch_types=[
            pltpu.VMEM_SHARED((n_acc, d), jnp.float32),
            pltpu.VMEM((2, CHUNK), jnp.int32),
            pltpu.VMEM((CHUNK, d), jnp.float32),
            pltpu.VMEM((64, d), jnp.float32),
        ],
    )
    def k(dst_hbm, out_hbm, acc, dstb, onesb, zb):
        cid = lax.axis_index("c")
        sid = lax.axis_index("s")
        wid = cid * NS + sid
        _fill(zb, 64, d, 0.0)
        _fill(onesb, CHUNK, d, 1.0)
        rps = n_acc // NS

        @pl.loop(0, rps // 64)
        def _(i):
            pltpu.sync_copy(zb, acc.at[pl.ds(sid * rps + i * 64, 64)])

        plsc.subcore_barrier()
        ew = n_chunks * CHUNK

        @pl.loop(0, n_chunks)
        def _(ci):
            off = wid * ew + ci * CHUNK
            pltpu.sync_copy(dst_hbm.at[pl.ds(off, CHUNK)], dstb.at[0])
            pltpu.sync_copy(onesb, acc.at[dstb.at[0]], add=True)

        plsc.subcore_barrier()

        @pl.loop(0, rps // 64)
        def _(i):
            r0 = sid * rps + i * 64
            pltpu.sync_copy(acc.at[pl.ds(r0, 64)], out_hbm.at[cid, pl.ds(r0, 64)])

    return k(dst_p)


def _sc_edge_pass(g, src_p, dst_p, n_acc, n_chunks):
    d = g.shape[1]

    @functools.partial(
        pl.kernel,
        out_type=jax.ShapeDtypeStruct((NC, n_acc, d), jnp.float32),
        mesh=_mesh(),
        scratch_types=[
            pltpu.VMEM_SHARED((n_acc, d), jnp.float32),
            pltpu.VMEM((2, CHUNK), jnp.int32),
            pltpu.VMEM((2, CHUNK), jnp.int32),
            pltpu.VMEM((2, CHUNK, d), jnp.float32),
            pltpu.VMEM((64, d), jnp.float32),
        ],
    )
    def k(g_hbm, src_hbm, dst_hbm, out_hbm, acc, srcb, dstb, rows, zb):
        cid = lax.axis_index("c")
        sid = lax.axis_index("s")
        wid = cid * NS + sid
        _fill(zb, 64, d, 0.0)
        rps = n_acc // NS

        @pl.loop(0, rps // 64)
        def _(i):
            pltpu.sync_copy(zb, acc.at[pl.ds(sid * rps + i * 64, 64)])

        plsc.subcore_barrier()
        ew = n_chunks * CHUNK

        @pl.loop(0, n_chunks)
        def _(ci):
            off = wid * ew + ci * CHUNK
            pltpu.sync_copy(src_hbm.at[pl.ds(off, CHUNK)], srcb.at[0])
            pltpu.sync_copy(dst_hbm.at[pl.ds(off, CHUNK)], dstb.at[0])
            pltpu.sync_copy(g_hbm.at[srcb.at[0]], rows.at[0])
            pltpu.sync_copy(rows.at[0], acc.at[dstb.at[0]], add=True)

        plsc.subcore_barrier()

        @pl.loop(0, rps // 64)
        def _(i):
            r0 = sid * rps + i * 64
            pltpu.sync_copy(acc.at[pl.ds(r0, 64)], out_hbm.at[cid, pl.ds(r0, 64)])

    return k(g, src_p, dst_p)


_DOT = dict(preferred_element_type=jnp.float32, precision=lax.Precision.HIGHEST)


def _tc_matmul(x, w):
    n, d = x.shape
    rb = 2000 if n % 2000 == 0 else n

    def body(x_ref, w_ref, o_ref):
        o_ref[...] = jnp.dot(x_ref[...], w_ref[...], **_DOT)

    return pl.pallas_call(
        body,
        grid=(n // rb,),
        in_specs=[
            pl.BlockSpec((rb, d), lambda i: (i, 0)),
            pl.BlockSpec((d, w.shape[1]), lambda i: (0, 0)),
        ],
        out_specs=pl.BlockSpec((rb, w.shape[1]), lambda i: (i, 0)),
        out_shape=jax.ShapeDtypeStruct((n, w.shape[1]), jnp.float32),
    )(x, w)


def _tc_scale(h1, degs):
    n, d = h1.shape
    rb = 2000 if n % 2000 == 0 else n

    def body(h_ref, degs_ref, g_ref, dinv_ref):
        deg = degs_ref[0] + degs_ref[1] + 1.0
        dinv = lax.rsqrt(jnp.maximum(deg, 1.0))
        dinv_ref[...] = dinv
        g_ref[...] = h_ref[...] * dinv

    return pl.pallas_call(
        body,
        grid=(n // rb,),
        in_specs=[
            pl.BlockSpec((rb, d), lambda i: (i, 0)),
            pl.BlockSpec((2, rb, d), lambda i: (0, i, 0)),
        ],
        out_specs=[
            pl.BlockSpec((rb, d), lambda i: (i, 0)),
            pl.BlockSpec((rb, d), lambda i: (i, 0)),
        ],
        out_shape=[
            jax.ShapeDtypeStruct((n, d), jnp.float32),
            jax.ShapeDtypeStruct((n, d), jnp.float32),
        ],
    )(h1, degs)


def _tc_mid(acc, g1, dinvb, b1, w2):
    n, d = g1.shape
    rb = 2000 if n % 2000 == 0 else n

    def body(acc_ref, g_ref, dinv_ref, b_ref, w_ref, o_ref):
        s = acc_ref[0] + acc_ref[1] + g_ref[...]
        h = jnp.maximum(dinv_ref[...] * s + b_ref[...], 0.0)
        o_ref[...] = jnp.dot(h, w_ref[...], **_DOT) * dinv_ref[...]

    return pl.pallas_call(
        body,
        grid=(n // rb,),
        in_specs=[
            pl.BlockSpec((2, rb, d), lambda i: (0, i, 0)),
            pl.BlockSpec((rb, d), lambda i: (i, 0)),
            pl.BlockSpec((rb, d), lambda i: (i, 0)),
            pl.BlockSpec((1, d), lambda i: (0, 0)),
            pl.BlockSpec((d, d), lambda i: (0, 0)),
        ],
        out_specs=pl.BlockSpec((rb, d), lambda i: (i, 0)),
        out_shape=jax.ShapeDtypeStruct((n, d), jnp.float32),
    )(acc, g1, dinvb, b1, w2)


def _tc_final(acc, g2, dinvb, b2, batch, wc, bc):
    n, d = g2.shape
    dout = wc.shape[1]
    rb = 2000 if n % 2000 == 0 else n
    nblk = n // rb
    batch3d = batch.reshape(nblk, 1, rb)

    def body(acc_ref, g_ref, dinv_ref, b_ref, batch_ref, wc_ref, bc_ref,
             o_ref, s_ref, c_ref):
        i = pl.program_id(0)

        @pl.when(i == 0)
        def _():
            s_ref[...] = jnp.zeros_like(s_ref)
            c_ref[...] = jnp.zeros_like(c_ref)

        s = acc_ref[0] + acc_ref[1] + g_ref[...]
        h = jnp.maximum(dinv_ref[...] * s + b_ref[...], 0.0)
        seg = lax.broadcasted_iota(jnp.int32, (NUM_GRAPHS, rb), 0)
        maskt = (seg == batch_ref[0]).astype(jnp.float32)
        s_ref[...] += jnp.dot(maskt, h, **_DOT)
        c_ref[...] = c_ref[...] + jnp.sum(maskt, axis=1, keepdims=True)

        @pl.when(i == nblk - 1)
        def _():
            pooled = s_ref[...] / jnp.maximum(c_ref[...], 1.0)
            o_ref[...] = jnp.dot(pooled, wc_ref[...], **_DOT) + bc_ref[...]

    return pl.pallas_call(
        body,
        grid=(nblk,),
        in_specs=[
            pl.BlockSpec((2, rb, d), lambda i: (0, i, 0)),
            pl.BlockSpec((rb, d), lambda i: (i, 0)),
            pl.BlockSpec((rb, d), lambda i: (i, 0)),
            pl.BlockSpec((1, d), lambda i: (0, 0)),
            pl.BlockSpec((1, 1, rb), lambda i: (i, 0, 0)),
            pl.BlockSpec((d, dout), lambda i: (0, 0)),
            pl.BlockSpec((1, dout), lambda i: (0, 0)),
        ],
        out_specs=pl.BlockSpec((NUM_GRAPHS, dout), lambda i: (0, 0)),
        out_shape=jax.ShapeDtypeStruct((NUM_GRAPHS, dout), jnp.float32),
        scratch_shapes=[
            pltpu.VMEM((NUM_GRAPHS, d), jnp.float32),
            pltpu.VMEM((NUM_GRAPHS, d), jnp.float32),
        ],
    )(acc, g2, dinvb, b2, batch3d, wc, bc)


def kernel(x, edge_index, batch, W1, b1, W2, b2, Wc, bc):
    n, d = x.shape
    src = edge_index[0].astype(jnp.int32)
    dst = edge_index[1].astype(jnp.int32)
    e = src.shape[0]

    n_acc = -(-n // 1024) * 1024
    n_chunks = -(-e // (NW * CHUNK))
    e_pad = NW * n_chunks * CHUNK
    npad = e_pad - e
    pad_idx = jnp.arange(npad, dtype=jnp.int32)
    src_p = jnp.concatenate([src, pad_idx % n])
    dst_p = jnp.concatenate([dst, n + pad_idx % (n_acc - n)])

    degs = _sc_degree(dst_p, n_acc, d, n_chunks)
    h1 = _tc_matmul(x, W1)
    g1, dinvb = _tc_scale(h1, degs)
    acc1 = _sc_edge_pass(g1, src_p, dst_p, n_acc, n_chunks)
    g2 = _tc_mid(acc1, g1, dinvb, b1.reshape(1, -1), W2)
    acc2 = _sc_edge_pass(g2, src_p, dst_p, n_acc, n_chunks)
    return _tc_final(acc2, g2, dinvb, b2.reshape(1, -1),
                     batch.astype(jnp.int32), Wc, bc.reshape(1, -1))


# speedup vs baseline: 24.9777x; 1.6228x over previous
import functools

import jax
import jax.numpy as jnp
from jax import lax
from jax.experimental import pallas as pl
from jax.experimental.pallas import tpu as pltpu
from jax.experimental.pallas import tpu_sc as plsc

NC = 2
NS = 16
NW = NC * NS
CHUNK = 128
NB_D = 4
NB_E = 2
NBR = 4
NUM_GRAPHS = 64


def _mesh():
    return plsc.VectorSubcoreMesh(core_axis_name="c", subcore_axis_name="s")


def _fill(ref, rows, d, value):
    @pl.loop(0, rows)
    def _(r):
        @pl.loop(0, d, step=16)
        def _(c):
            ref[r, pl.ds(c, 16)] = jnp.full((16,), value, jnp.float32)


def _sc_degree(dst_p, n_acc, d, n_chunks):
    @functools.partial(
        pl.kernel,
        out_type=jax.ShapeDtypeStruct((NC, n_acc, d), jnp.float32),
        mesh=_mesh(),
        scratch_types=[
            pltpu.VMEM_SHARED((n_acc, d), jnp.float32),
            pltpu.VMEM((NB_D, CHUNK), jnp.int32),
            pltpu.VMEM((CHUNK, d), jnp.float32),
            pltpu.VMEM((16, d), jnp.float32),
            pltpu.SemaphoreType.DMA((NB_D,)),
            pltpu.SemaphoreType.DMA((NB_D,)),
        ],
    )
    def k(dst_hbm, out_hbm, acc, dstb, onesb, zb, si, ss):
        cid = lax.axis_index("c")
        sid = lax.axis_index("s")
        wid = cid * NS + sid
        _fill(zb, 16, d, 0.0)
        _fill(onesb, CHUNK, d, 1.0)
        rps = n_acc // NS

        @pl.loop(0, rps // 16)
        def _(i):
            pltpu.sync_copy(zb, acc.at[pl.ds(sid * rps + i * 16, 16)])

        plsc.subcore_barrier()
        base = wid * (n_chunks * CHUNK)

        def idx_start(b, ci):
            off = base + ci * CHUNK
            pltpu.async_copy(dst_hbm.at[pl.ds(off, CHUNK)], dstb.at[b], si.at[b])

        def idx_wait(b, ci):
            off = base + ci * CHUNK
            pltpu.make_async_copy(
                dst_hbm.at[pl.ds(off, CHUNK)], dstb.at[b], si.at[b]).wait()

        def scat_start(b):
            pltpu.async_copy(onesb, acc.at[dstb.at[b]], ss.at[b], add=True)

        def scat_wait(b):
            pltpu.make_async_copy(onesb, acc.at[dstb.at[b]], ss.at[b]).wait()

        for b in range(NB_D):
            idx_start(b, b)

        @pl.loop(0, n_chunks // NB_D)
        def _(q):
            for b in range(NB_D):
                ci = q * NB_D + b

                @pl.when(q > 0)
                def _():
                    scat_wait(b)
                    idx_start(b, ci)

                idx_wait(b, ci)
                scat_start(b)

        for b in range(NB_D):
            scat_wait(b)

        plsc.subcore_barrier()

        @pl.loop(0, rps // 64)
        def _(i):
            r0 = sid * rps + i * 64
            pltpu.sync_copy(acc.at[pl.ds(r0, 64)], out_hbm.at[cid, pl.ds(r0, 64)])

    return k(dst_p)


def _sc_edge_pass(g, src_p, dst_p, n_acc, n_chunks):
    d = g.shape[1]

    @functools.partial(
        pl.kernel,
        out_type=jax.ShapeDtypeStruct((NC, n_acc, d), jnp.float32),
        mesh=_mesh(),
        scratch_types=[
            pltpu.VMEM_SHARED((n_acc, d), jnp.float32),
            pltpu.VMEM((NB_E, CHUNK), jnp.int32),
            pltpu.VMEM((NB_E, CHUNK), jnp.int32),
            pltpu.VMEM((NB_E, CHUNK, d), jnp.float32),
            pltpu.VMEM((16, d), jnp.float32),
            pltpu.SemaphoreType.DMA((NB_E,)),
            pltpu.SemaphoreType.DMA((NB_E,)),
            pltpu.SemaphoreType.DMA((NB_E,)),
        ],
    )
    def k(g_hbm, src_hbm, dst_hbm, out_hbm, acc, srcb, dstb, rows, zb,
          si, sg, ss):
        cid = lax.axis_index("c")
        sid = lax.axis_index("s")
        wid = cid * NS + sid
        _fill(zb, 16, d, 0.0)
        rps = n_acc // NS

        @pl.loop(0, rps // 16)
        def _(i):
            pltpu.sync_copy(zb, acc.at[pl.ds(sid * rps + i * 16, 16)])

        plsc.subcore_barrier()
        base = wid * (n_chunks * CHUNK)

        def idx_start(b, ci):
            off = base + ci * CHUNK
            pltpu.async_copy(src_hbm.at[pl.ds(off, CHUNK)], srcb.at[b], si.at[b])
            pltpu.async_copy(dst_hbm.at[pl.ds(off, CHUNK)], dstb.at[b], si.at[b])

        def idx_wait(b, ci):
            off = base + ci * CHUNK
            pltpu.make_async_copy(
                src_hbm.at[pl.ds(off, CHUNK)], srcb.at[b], si.at[b]).wait()
            pltpu.make_async_copy(
                dst_hbm.at[pl.ds(off, CHUNK)], dstb.at[b], si.at[b]).wait()

        def gather_start(b):
            pltpu.async_copy(g_hbm.at[srcb.at[b]], rows.at[b], sg.at[b])

        def gather_wait(b):
            pltpu.make_async_copy(g_hbm.at[srcb.at[b]], rows.at[b],
                                  sg.at[b]).wait()

        def scat_start(b):
            pltpu.async_copy(rows.at[b], acc.at[dstb.at[b]], ss.at[b], add=True)

        def scat_wait(b):
            pltpu.make_async_copy(rows.at[b], acc.at[dstb.at[b]],
                                  ss.at[b]).wait()

        for b in range(NB_E):
            idx_start(b, b)

        @pl.loop(0, n_chunks // NB_E)
        def _(q):
            for b in range(NB_E):
                ci = q * NB_E + b

                @pl.when(q > 0)
                def _():
                    scat_wait(b)
                    idx_start(b, ci)

                idx_wait(b, ci)
                gather_start(b)

            for b in range(NB_E):
                gather_wait(b)
                scat_start(b)

        for b in range(NB_E):
            scat_wait(b)

        plsc.subcore_barrier()

        @pl.loop(0, rps // 64)
        def _(i):
            r0 = sid * rps + i * 64
            pltpu.sync_copy(acc.at[pl.ds(r0, 64)], out_hbm.at[cid, pl.ds(r0, 64)])

    return k(g, src_p, dst_p)


_DOT = dict(preferred_element_type=jnp.float32, precision=lax.Precision.HIGHEST)


def _tc_matmul(x, w):
    n, d = x.shape
    rb = 2000 if n % 2000 == 0 else n

    def body(x_ref, w_ref, o_ref):
        o_ref[...] = jnp.dot(x_ref[...], w_ref[...], **_DOT)

    return pl.pallas_call(
        body,
        grid=(n // rb,),
        in_specs=[
            pl.BlockSpec((rb, d), lambda i: (i, 0)),
            pl.BlockSpec((d, w.shape[1]), lambda i: (0, 0)),
        ],
        out_specs=pl.BlockSpec((rb, w.shape[1]), lambda i: (i, 0)),
        out_shape=jax.ShapeDtypeStruct((n, w.shape[1]), jnp.float32),
    )(x, w)


def _tc_scale(h1, degs):
    n, d = h1.shape
    rb = 2000 if n % 2000 == 0 else n

    def body(h_ref, degs_ref, g_ref, dinv_ref):
        deg = degs_ref[0] + degs_ref[1] + 1.0
        dinv = lax.rsqrt(jnp.maximum(deg, 1.0))
        dinv_ref[...] = dinv
        g_ref[...] = h_ref[...] * dinv

    return pl.pallas_call(
        body,
        grid=(n // rb,),
        in_specs=[
            pl.BlockSpec((rb, d), lambda i: (i, 0)),
            pl.BlockSpec((2, rb, d), lambda i: (0, i, 0)),
        ],
        out_specs=[
            pl.BlockSpec((rb, d), lambda i: (i, 0)),
            pl.BlockSpec((rb, d), lambda i: (i, 0)),
        ],
        out_shape=[
            jax.ShapeDtypeStruct((n, d), jnp.float32),
            jax.ShapeDtypeStruct((n, d), jnp.float32),
        ],
    )(h1, degs)


def _tc_mid(acc, g1, dinvb, b1, w2):
    n, d = g1.shape
    rb = 2000 if n % 2000 == 0 else n

    def body(acc_ref, g_ref, dinv_ref, b_ref, w_ref, o_ref):
        s = acc_ref[0] + acc_ref[1] + g_ref[...]
        h = jnp.maximum(dinv_ref[...] * s + b_ref[...], 0.0)
        o_ref[...] = jnp.dot(h, w_ref[...], **_DOT) * dinv_ref[...]

    return pl.pallas_call(
        body,
        grid=(n // rb,),
        in_specs=[
            pl.BlockSpec((2, rb, d), lambda i: (0, i, 0)),
            pl.BlockSpec((rb, d), lambda i: (i, 0)),
            pl.BlockSpec((rb, d), lambda i: (i, 0)),
            pl.BlockSpec((1, d), lambda i: (0, 0)),
            pl.BlockSpec((d, d), lambda i: (0, 0)),
        ],
        out_specs=pl.BlockSpec((rb, d), lambda i: (i, 0)),
        out_shape=jax.ShapeDtypeStruct((n, d), jnp.float32),
    )(acc, g1, dinvb, b1, w2)


def _tc_final(acc, g2, dinvb, b2, batch, wc, bc):
    n, d = g2.shape
    dout = wc.shape[1]
    rb = 2000 if n % 2000 == 0 else n
    nblk = n // rb
    batch3d = batch.reshape(nblk, 1, rb)

    def body(acc_ref, g_ref, dinv_ref, b_ref, batch_ref, wc_ref, bc_ref,
             o_ref, s_ref, c_ref):
        i = pl.program_id(0)

        @pl.when(i == 0)
        def _():
            s_ref[...] = jnp.zeros_like(s_ref)
            c_ref[...] = jnp.zeros_like(c_ref)

        s = acc_ref[0] + acc_ref[1] + g_ref[...]
        h = jnp.maximum(dinv_ref[...] * s + b_ref[...], 0.0)
        seg = lax.broadcasted_iota(jnp.int32, (NUM_GRAPHS, rb), 0)
        maskt = (seg == batch_ref[0]).astype(jnp.float32)
        s_ref[...] += jnp.dot(maskt, h, **_DOT)
        c_ref[...] = c_ref[...] + jnp.sum(maskt, axis=1, keepdims=True)

        @pl.when(i == nblk - 1)
        def _():
            pooled = s_ref[...] / jnp.maximum(c_ref[...], 1.0)
            o_ref[...] = jnp.dot(pooled, wc_ref[...], **_DOT) + bc_ref[...]

    return pl.pallas_call(
        body,
        grid=(nblk,),
        in_specs=[
            pl.BlockSpec((2, rb, d), lambda i: (0, i, 0)),
            pl.BlockSpec((rb, d), lambda i: (i, 0)),
            pl.BlockSpec((rb, d), lambda i: (i, 0)),
            pl.BlockSpec((1, d), lambda i: (0, 0)),
            pl.BlockSpec((1, 1, rb), lambda i: (i, 0, 0)),
            pl.BlockSpec((d, dout), lambda i: (0, 0)),
            pl.BlockSpec((1, dout), lambda i: (0, 0)),
        ],
        out_specs=pl.BlockSpec((NUM_GRAPHS, dout), lambda i: (0, 0)),
        out_shape=jax.ShapeDtypeStruct((NUM_GRAPHS, dout), jnp.float32),
        scratch_shapes=[
            pltpu.VMEM((NUM_GRAPHS, d), jnp.float32),
            pltpu.VMEM((NUM_GRAPHS, d), jnp.float32),
        ],
    )(acc, g2, dinvb, b2, batch3d, wc, bc)


def kernel(x, edge_index, batch, W1, b1, W2, b2, Wc, bc):
    n, d = x.shape
    src = edge_index[0].astype(jnp.int32)
    dst = edge_index[1].astype(jnp.int32)
    e = src.shape[0]

    n_acc = -(-n // 1024) * 1024
    n_chunks = -(-e // (NW * CHUNK * NBR)) * NBR
    e_pad = NW * n_chunks * CHUNK
    npad = e_pad - e
    pad_idx = jnp.arange(npad, dtype=jnp.int32)
    src_p = jnp.concatenate([src, pad_idx % n])
    dst_p = jnp.concatenate([dst, n + pad_idx % (n_acc - n)])

    degs = _sc_degree(dst_p, n_acc, d, n_chunks)
    h1 = _tc_matmul(x, W1)
    g1, dinvb = _tc_scale(h1, degs)
    acc1 = _sc_edge_pass(g1, src_p, dst_p, n_acc, n_chunks)
    g2 = _tc_mid(acc1, g1, dinvb, b1.reshape(1, -1), W2)
    acc2 = _sc_edge_pass(g2, src_p, dst_p, n_acc, n_chunks)
    return _tc_final(acc2, g2, dinvb, b2.reshape(1, -1),
                     batch.astype(jnp.int32), Wc, bc.reshape(1, -1))


# speedup vs baseline: 28.8303x; 1.1542x over previous
import functools

import jax
import jax.numpy as jnp
from jax import lax
from jax.experimental import pallas as pl
from jax.experimental.pallas import tpu as pltpu
from jax.experimental.pallas import tpu_sc as plsc

NC = 2
NS = 16
NW = NC * NS
CHUNK = 128
NB_D = 4
NB_E = 2
NBR = 4
NUM_GRAPHS = 64


def _mesh():
    return plsc.VectorSubcoreMesh(core_axis_name="c", subcore_axis_name="s")


def _fill(ref, rows, d, value):
    @pl.loop(0, rows)
    def _(r):
        @pl.loop(0, d, step=16)
        def _(c):
            ref[r, pl.ds(c, 16)] = jnp.full((16,), value, jnp.float32)


def _sc_degree(dst_p, n_acc, d, n_chunks):
    @functools.partial(
        pl.kernel,
        out_type=jax.ShapeDtypeStruct((NC, n_acc, d), jnp.float32),
        mesh=_mesh(),
        scratch_types=[
            pltpu.VMEM_SHARED((n_acc, d), jnp.float32),
            pltpu.VMEM((4, CHUNK), jnp.int32),
            pltpu.VMEM((CHUNK, d), jnp.float32),
            pltpu.VMEM((16, d), jnp.float32),
            pltpu.SemaphoreType.DMA((4,)),
            pltpu.SemaphoreType.DMA((4,)),
        ],
    )
    def k(dst_hbm, out_hbm, acc, dstb, onesb, zb, si, ss):
        cid = lax.axis_index("c")
        sid = lax.axis_index("s")
        wid = cid * NS + sid
        _fill(zb, 16, d, 0.0)
        _fill(onesb, CHUNK, d, 1.0)
        rps = n_acc // NS

        @pl.loop(0, rps // 16)
        def _(i):
            pltpu.sync_copy(zb, acc.at[pl.ds(sid * rps + i * 16, 16)])

        plsc.subcore_barrier()
        base = wid * (n_chunks * CHUNK)

        def idx_start(s, ci):
            pltpu.async_copy(dst_hbm.at[pl.ds(base + ci * CHUNK, CHUNK)],
                             dstb.at[s], si.at[s])

        def idx_wait(s, ci):
            pltpu.make_async_copy(dst_hbm.at[pl.ds(base + ci * CHUNK, CHUNK)],
                                  dstb.at[s], si.at[s]).wait()

        def scat_start(s):
            pltpu.async_copy(onesb, acc.at[dstb.at[s]], ss.at[s], add=True)

        def scat_wait(s):
            pltpu.make_async_copy(onesb, acc.at[dstb.at[s]], ss.at[s]).wait()

        for s in range(4):
            idx_start(s, s)

        @pl.loop(0, n_chunks // 4)
        def _(t):
            for j in range(4):
                ci = t * 4 + j
                sp = (j + 2) % 4

                def pre():
                    scat_wait(sp)

                    @pl.when(ci + 2 < n_chunks)
                    def _():
                        idx_start(sp, ci + 2)

                if j >= 2:
                    pre()
                else:
                    @pl.when(t > 0)
                    def _():
                        pre()

                idx_wait(j, ci)
                scat_start(j)

        for s in (2, 3):
            scat_wait(s)

        plsc.subcore_barrier()

        @pl.loop(0, rps // 64)
        def _(i):
            r0 = sid * rps + i * 64
            pltpu.sync_copy(acc.at[pl.ds(r0, 64)], out_hbm.at[cid, pl.ds(r0, 64)])

    return k(dst_p)


def _sc_edge_pass(g, src_p, dst_p, n_acc, n_chunks):
    d = g.shape[1]

    @functools.partial(
        pl.kernel,
        out_type=jax.ShapeDtypeStruct((NC, n_acc, d), jnp.float32),
        mesh=_mesh(),
        scratch_types=[
            pltpu.VMEM_SHARED((n_acc, d), jnp.float32),
            pltpu.VMEM((4, CHUNK), jnp.int32),
            pltpu.VMEM((4, CHUNK), jnp.int32),
            pltpu.VMEM((2, CHUNK, d), jnp.float32),
            pltpu.VMEM((16, d), jnp.float32),
            pltpu.SemaphoreType.DMA((4,)),
            pltpu.SemaphoreType.DMA((2,)),
            pltpu.SemaphoreType.DMA((4,)),
        ],
    )
    def k(g_hbm, src_hbm, dst_hbm, out_hbm, acc, srcb, dstb, rows, zb,
          si, sg, ss):
        cid = lax.axis_index("c")
        sid = lax.axis_index("s")
        wid = cid * NS + sid
        _fill(zb, 16, d, 0.0)
        rps = n_acc // NS

        @pl.loop(0, rps // 16)
        def _(i):
            pltpu.sync_copy(zb, acc.at[pl.ds(sid * rps + i * 16, 16)])

        plsc.subcore_barrier()
        base = wid * (n_chunks * CHUNK)

        def idx_start(s, ci):
            off = base + ci * CHUNK
            pltpu.async_copy(src_hbm.at[pl.ds(off, CHUNK)], srcb.at[s], si.at[s])
            pltpu.async_copy(dst_hbm.at[pl.ds(off, CHUNK)], dstb.at[s], si.at[s])

        def idx_wait(s, ci):
            off = base + ci * CHUNK
            pltpu.make_async_copy(
                src_hbm.at[pl.ds(off, CHUNK)], srcb.at[s], si.at[s]).wait()
            pltpu.make_async_copy(
                dst_hbm.at[pl.ds(off, CHUNK)], dstb.at[s], si.at[s]).wait()

        def gather_start(r, s):
            pltpu.async_copy(g_hbm.at[srcb.at[s]], rows.at[r], sg.at[r])

        def gather_wait(r, s):
            pltpu.make_async_copy(g_hbm.at[srcb.at[s]], rows.at[r],
                                  sg.at[r]).wait()

        def scat_start(r, s):
            pltpu.async_copy(rows.at[r], acc.at[dstb.at[s]], ss.at[s], add=True)

        def scat_wait(r, s):
            pltpu.make_async_copy(rows.at[r], acc.at[dstb.at[s]],
                                  ss.at[s]).wait()

        for s in range(4):
            idx_start(s, s)

        @pl.loop(0, n_chunks // 4)
        def _(t):
            for j in range(4):
                ci = t * 4 + j
                sp = (j + 2) % 4
                sv = (j + 3) % 4

                def pre():
                    scat_wait(j % 2, sp)

                    @pl.when(ci + 2 < n_chunks)
                    def _():
                        idx_start(sp, ci + 2)

                if j >= 2:
                    pre()
                else:
                    @pl.when(t > 0)
                    def _():
                        pre()

                idx_wait(j, ci)
                gather_start(j % 2, j)

                def flow():
                    gather_wait((j + 1) % 2, sv)
                    scat_start((j + 1) % 2, sv)

                if j >= 1:
                    flow()
                else:
                    @pl.when(t > 0)
                    def _():
                        flow()

        gather_wait(1, 3)
        scat_start(1, 3)
        scat_wait(0, 2)
        scat_wait(1, 3)

        plsc.subcore_barrier()

        @pl.loop(0, rps // 64)
        def _(i):
            r0 = sid * rps + i * 64
            pltpu.sync_copy(acc.at[pl.ds(r0, 64)], out_hbm.at[cid, pl.ds(r0, 64)])

    return k(g, src_p, dst_p)


_DOT = dict(preferred_element_type=jnp.float32, precision=lax.Precision.HIGHEST)


def _tc_matmul(x, w):
    n, d = x.shape
    rb = 2000 if n % 2000 == 0 else n

    def body(x_ref, w_ref, o_ref):
        o_ref[...] = jnp.dot(x_ref[...], w_ref[...], **_DOT)

    return pl.pallas_call(
        body,
        grid=(n // rb,),
        in_specs=[
            pl.BlockSpec((rb, d), lambda i: (i, 0)),
            pl.BlockSpec((d, w.shape[1]), lambda i: (0, 0)),
        ],
        out_specs=pl.BlockSpec((rb, w.shape[1]), lambda i: (i, 0)),
        out_shape=jax.ShapeDtypeStruct((n, w.shape[1]), jnp.float32),
    )(x, w)


def _tc_scale(h1, degs):
    n, d = h1.shape
    rb = 2000 if n % 2000 == 0 else n

    def body(h_ref, degs_ref, g_ref, dinv_ref):
        deg = degs_ref[0] + degs_ref[1] + 1.0
        dinv = lax.rsqrt(jnp.maximum(deg, 1.0))
        dinv_ref[...] = dinv
        g_ref[...] = h_ref[...] * dinv

    return pl.pallas_call(
        body,
        grid=(n // rb,),
        in_specs=[
            pl.BlockSpec((rb, d), lambda i: (i, 0)),
            pl.BlockSpec((2, rb, d), lambda i: (0, i, 0)),
        ],
        out_specs=[
            pl.BlockSpec((rb, d), lambda i: (i, 0)),
            pl.BlockSpec((rb, d), lambda i: (i, 0)),
        ],
        out_shape=[
            jax.ShapeDtypeStruct((n, d), jnp.float32),
            jax.ShapeDtypeStruct((n, d), jnp.float32),
        ],
    )(h1, degs)


def _tc_mid(acc, g1, dinvb, b1, w2):
    n, d = g1.shape
    rb = 2000 if n % 2000 == 0 else n

    def body(acc_ref, g_ref, dinv_ref, b_ref, w_ref, o_ref):
        s = acc_ref[0] + acc_ref[1] + g_ref[...]
        h = jnp.maximum(dinv_ref[...] * s + b_ref[...], 0.0)
        o_ref[...] = jnp.dot(h, w_ref[...], **_DOT) * dinv_ref[...]

    return pl.pallas_call(
        body,
        grid=(n // rb,),
        in_specs=[
            pl.BlockSpec((2, rb, d), lambda i: (0, i, 0)),
            pl.BlockSpec((rb, d), lambda i: (i, 0)),
            pl.BlockSpec((rb, d), lambda i: (i, 0)),
            pl.BlockSpec((1, d), lambda i: (0, 0)),
            pl.BlockSpec((d, d), lambda i: (0, 0)),
        ],
        out_specs=pl.BlockSpec((rb, d), lambda i: (i, 0)),
        out_shape=jax.ShapeDtypeStruct((n, d), jnp.float32),
    )(acc, g1, dinvb, b1, w2)


def _tc_final(acc, g2, dinvb, b2, batch, wc, bc):
    n, d = g2.shape
    dout = wc.shape[1]
    rb = 2000 if n % 2000 == 0 else n
    nblk = n // rb
    batch3d = batch.reshape(nblk, 1, rb)

    def body(acc_ref, g_ref, dinv_ref, b_ref, batch_ref, wc_ref, bc_ref,
             o_ref, s_ref, c_ref):
        i = pl.program_id(0)

        @pl.when(i == 0)
        def _():
            s_ref[...] = jnp.zeros_like(s_ref)
            c_ref[...] = jnp.zeros_like(c_ref)

        s = acc_ref[0] + acc_ref[1] + g_ref[...]
        h = jnp.maximum(dinv_ref[...] * s + b_ref[...], 0.0)
        seg = lax.broadcasted_iota(jnp.int32, (NUM_GRAPHS, rb), 0)
        maskt = (seg == batch_ref[0]).astype(jnp.float32)
        s_ref[...] += jnp.dot(maskt, h, **_DOT)
        c_ref[...] = c_ref[...] + jnp.sum(maskt, axis=1, keepdims=True)

        @pl.when(i == nblk - 1)
        def _():
            pooled = s_ref[...] / jnp.maximum(c_ref[...], 1.0)
            o_ref[...] = jnp.dot(pooled, wc_ref[...], **_DOT) + bc_ref[...]

    return pl.pallas_call(
        body,
        grid=(nblk,),
        in_specs=[
            pl.BlockSpec((2, rb, d), lambda i: (0, i, 0)),
            pl.BlockSpec((rb, d), lambda i: (i, 0)),
            pl.BlockSpec((rb, d), lambda i: (i, 0)),
            pl.BlockSpec((1, d), lambda i: (0, 0)),
            pl.BlockSpec((1, 1, rb), lambda i: (i, 0, 0)),
            pl.BlockSpec((d, dout), lambda i: (0, 0)),
            pl.BlockSpec((1, dout), lambda i: (0, 0)),
        ],
        out_specs=pl.BlockSpec((NUM_GRAPHS, dout), lambda i: (0, 0)),
        out_shape=jax.ShapeDtypeStruct((NUM_GRAPHS, dout), jnp.float32),
        scratch_shapes=[
            pltpu.VMEM((NUM_GRAPHS, d), jnp.float32),
            pltpu.VMEM((NUM_GRAPHS, d), jnp.float32),
        ],
    )(acc, g2, dinvb, b2, batch3d, wc, bc)


def kernel(x, edge_index, batch, W1, b1, W2, b2, Wc, bc):
    n, d = x.shape
    src = edge_index[0].astype(jnp.int32)
    dst = edge_index[1].astype(jnp.int32)
    e = src.shape[0]

    n_acc = -(-n // 1024) * 1024
    n_chunks = -(-e // (NW * CHUNK * NBR)) * NBR
    e_pad = NW * n_chunks * CHUNK
    npad = e_pad - e
    pad_idx = jnp.arange(npad, dtype=jnp.int32)
    src_p = jnp.concatenate([src, pad_idx % n])
    dst_p = jnp.concatenate([dst, n + pad_idx % (n_acc - n)])

    degs = _sc_degree(dst_p, n_acc, d, n_chunks)
    h1 = _tc_matmul(x, W1)
    g1, dinvb = _tc_scale(h1, degs)
    acc1 = _sc_edge_pass(g1, src_p, dst_p, n_acc, n_chunks)
    g2 = _tc_mid(acc1, g1, dinvb, b1.reshape(1, -1), W2)
    acc2 = _sc_edge_pass(g2, src_p, dst_p, n_acc, n_chunks)
    return _tc_final(acc2, g2, dinvb, b2.reshape(1, -1),
                     batch.astype(jnp.int32), Wc, bc.reshape(1, -1))


# speedup vs baseline: 33.7238x; 1.1697x over previous
import dataclasses
import functools

import jax
import jax.numpy as jnp
from jax import lax
from jax.experimental import pallas as pl
from jax.experimental.pallas import tpu as pltpu
from jax.experimental.pallas import tpu_sc as plsc

NC = 2
NS = 16
NW = NC * NS
CHUNK = 128
NB_D = 4
NB_E = 2
NBR = 4
NUM_GRAPHS = 64


def _mesh():
    return plsc.VectorSubcoreMesh(core_axis_name="c", subcore_axis_name="s")


def _sc_params():
    cp = pltpu.CompilerParams()
    if "needs_layout_passes" in pltpu.CompilerParams.__dataclass_fields__:
        cp = dataclasses.replace(cp, needs_layout_passes=False)
    return cp


def _fill(ref, rows, d, value):
    @pl.loop(0, rows)
    def _(r):
        @pl.loop(0, d, step=16)
        def _(c):
            ref[r, pl.ds(c, 16)] = jnp.full((16,), value, jnp.float32)


def _sc_degree(dst_p, n_acc, n_chunks):
    hr = n_acc // 128

    @functools.partial(
        pl.kernel,
        out_type=jax.ShapeDtypeStruct((NW, hr, 128), jnp.float32),
        mesh=_mesh(),
        scratch_types=[
            pltpu.VMEM((hr, 128), jnp.float32),
            pltpu.VMEM((4, CHUNK), jnp.int32),
            pltpu.SemaphoreType.DMA((4,)),
        ],
        compiler_params=_sc_params(),
    )
    def k(dst_hbm, out_hbm, hist, dstb, si):
        cid = lax.axis_index("c")
        sid = lax.axis_index("s")
        wid = cid * NS + sid
        _fill(hist, hr, 128, 0.0)
        base = wid * (n_chunks * CHUNK)

        def idx_start(s, ci):
            pltpu.async_copy(dst_hbm.at[pl.ds(base + ci * CHUNK, CHUNK)],
                             dstb.at[s], si.at[s])

        def idx_wait(s, ci):
            pltpu.make_async_copy(dst_hbm.at[pl.ds(base + ci * CHUNK, CHUNK)],
                                  dstb.at[s], si.at[s]).wait()

        for s in range(4):
            idx_start(s, s)

        @pl.loop(0, n_chunks // 4)
        def _(t):
            for j in range(4):
                ci = t * 4 + j
                idx_wait(j, ci)
                for g in range(CHUNK // 16):
                    v = dstb[j, pl.ds(g * 16, 16)]
                    cnt, last = plsc.scan_count(v)
                    plsc.addupdate_scatter(
                        hist,
                        [jax.lax.shift_right_logical(v, 7),
                         jax.lax.bitwise_and(v, 127)],
                        cnt.astype(jnp.float32), mask=last)

                @pl.when(ci + 4 < n_chunks)
                def _():
                    idx_start(j, ci + 4)

        pltpu.sync_copy(hist, out_hbm.at[wid])

    return k(dst_p)


def _sc_edge_pass(g, src_p, dst_p, n_acc, n_chunks):
    d = g.shape[1]

    @functools.partial(
        pl.kernel,
        out_type=jax.ShapeDtypeStruct((NC, n_acc, d), jnp.float32),
        mesh=_mesh(),
        scratch_types=[
            pltpu.VMEM_SHARED((n_acc, d), jnp.float32),
            pltpu.VMEM((4, CHUNK), jnp.int32),
            pltpu.VMEM((4, CHUNK), jnp.int32),
            pltpu.VMEM((2, CHUNK, d), jnp.float32),
            pltpu.VMEM((16, d), jnp.float32),
            pltpu.SemaphoreType.DMA((4,)),
            pltpu.SemaphoreType.DMA((2,)),
            pltpu.SemaphoreType.DMA((4,)),
        ],
    )
    def k(g_hbm, src_hbm, dst_hbm, out_hbm, acc, srcb, dstb, rows, zb,
          si, sg, ss):
        cid = lax.axis_index("c")
        sid = lax.axis_index("s")
        wid = cid * NS + sid
        _fill(zb, 16, d, 0.0)
        rps = n_acc // NS

        @pl.loop(0, rps // 16)
        def _(i):
            pltpu.sync_copy(zb, acc.at[pl.ds(sid * rps + i * 16, 16)])

        plsc.subcore_barrier()
        base = wid * (n_chunks * CHUNK)

        def idx_start(s, ci):
            off = base + ci * CHUNK
            pltpu.async_copy(src_hbm.at[pl.ds(off, CHUNK)], srcb.at[s], si.at[s])
            pltpu.async_copy(dst_hbm.at[pl.ds(off, CHUNK)], dstb.at[s], si.at[s])

        def idx_wait(s, ci):
            off = base + ci * CHUNK
            pltpu.make_async_copy(
                src_hbm.at[pl.ds(off, CHUNK)], srcb.at[s], si.at[s]).wait()
            pltpu.make_async_copy(
                dst_hbm.at[pl.ds(off, CHUNK)], dstb.at[s], si.at[s]).wait()

        def gather_start(r, s):
            pltpu.async_copy(g_hbm.at[srcb.at[s]], rows.at[r], sg.at[r])

        def gather_wait(r, s):
            pltpu.make_async_copy(g_hbm.at[srcb.at[s]], rows.at[r],
                                  sg.at[r]).wait()

        def scat_start(r, s):
            pltpu.async_copy(rows.at[r], acc.at[dstb.at[s]], ss.at[s], add=True)

        def scat_wait(r, s):
            pltpu.make_async_copy(rows.at[r], acc.at[dstb.at[s]],
                                  ss.at[s]).wait()

        for s in range(4):
            idx_start(s, s)

        @pl.loop(0, n_chunks // 4)
        def _(t):
            for j in range(4):
                ci = t * 4 + j
                sp = (j + 2) % 4
                sv = (j + 3) % 4

                def pre():
                    scat_wait(j % 2, sp)

                    @pl.when(ci + 2 < n_chunks)
                    def _():
                        idx_start(sp, ci + 2)

                if j >= 2:
                    pre()
                else:
                    @pl.when(t > 0)
                    def _():
                        pre()

                idx_wait(j, ci)
                gather_start(j % 2, j)

                def flow():
                    gather_wait((j + 1) % 2, sv)
                    scat_start((j + 1) % 2, sv)

                if j >= 1:
                    flow()
                else:
                    @pl.when(t > 0)
                    def _():
                        flow()

        gather_wait(1, 3)
        scat_start(1, 3)
        scat_wait(0, 2)
        scat_wait(1, 3)

        plsc.subcore_barrier()

        @pl.loop(0, rps // 64)
        def _(i):
            r0 = sid * rps + i * 64
            pltpu.sync_copy(acc.at[pl.ds(r0, 64)], out_hbm.at[cid, pl.ds(r0, 64)])

    return k(g, src_p, dst_p)


_DOT = dict(preferred_element_type=jnp.float32, precision=lax.Precision.HIGHEST)


def _tc_matmul(x, w):
    n, d = x.shape
    rb = 2000 if n % 2000 == 0 else n

    def body(x_ref, w_ref, o_ref):
        o_ref[...] = jnp.dot(x_ref[...], w_ref[...], **_DOT)

    return pl.pallas_call(
        body,
        grid=(n // rb,),
        in_specs=[
            pl.BlockSpec((rb, d), lambda i: (i, 0)),
            pl.BlockSpec((d, w.shape[1]), lambda i: (0, 0)),
        ],
        out_specs=pl.BlockSpec((rb, w.shape[1]), lambda i: (i, 0)),
        out_shape=jax.ShapeDtypeStruct((n, w.shape[1]), jnp.float32),
    )(x, w)


def _tc_degsum(degs, n_acc):
    hr = n_acc // 128

    def body(degs_ref, o_ref):
        deg = jnp.sum(degs_ref[...], axis=0) + 1.0
        o_ref[...] = lax.rsqrt(jnp.maximum(deg, 1.0))

    return pl.pallas_call(
        body,
        out_shape=jax.ShapeDtypeStruct((hr, 128), jnp.float32),
    )(degs)


def _tc_scale(h1, dinv_col):
    n, d = h1.shape
    rb = 2000 if n % 2000 == 0 else n

    def body(h_ref, dinv_ref, g_ref):
        g_ref[...] = h_ref[...] * dinv_ref[...]

    return pl.pallas_call(
        body,
        grid=(n // rb,),
        in_specs=[
            pl.BlockSpec((rb, d), lambda i: (i, 0)),
            pl.BlockSpec((rb, 1), lambda i: (i, 0)),
        ],
        out_specs=pl.BlockSpec((rb, d), lambda i: (i, 0)),
        out_shape=jax.ShapeDtypeStruct((n, d), jnp.float32),
    )(h1, dinv_col)


def _tc_mid(acc, g1, dinv_col, b1, w2):
    n, d = g1.shape
    rb = 2000 if n % 2000 == 0 else n

    def body(acc_ref, g_ref, dinv_ref, b_ref, w_ref, o_ref):
        s = acc_ref[0] + acc_ref[1] + g_ref[...]
        h = jnp.maximum(dinv_ref[...] * s + b_ref[...], 0.0)
        o_ref[...] = jnp.dot(h, w_ref[...], **_DOT) * dinv_ref[...]

    return pl.pallas_call(
        body,
        grid=(n // rb,),
        in_specs=[
            pl.BlockSpec((2, rb, d), lambda i: (0, i, 0)),
            pl.BlockSpec((rb, d), lambda i: (i, 0)),
            pl.BlockSpec((rb, 1), lambda i: (i, 0)),
            pl.BlockSpec((1, d), lambda i: (0, 0)),
            pl.BlockSpec((d, d), lambda i: (0, 0)),
        ],
        out_specs=pl.BlockSpec((rb, d), lambda i: (i, 0)),
        out_shape=jax.ShapeDtypeStruct((n, d), jnp.float32),
    )(acc, g1, dinv_col, b1, w2)


def _tc_final(acc, g2, dinv_col, b2, batch, wc, bc):
    n, d = g2.shape
    dout = wc.shape[1]
    rb = 2000 if n % 2000 == 0 else n
    nblk = n // rb
    batch3d = batch.reshape(nblk, 1, rb)

    def body(acc_ref, g_ref, dinv_ref, b_ref, batch_ref, wc_ref, bc_ref,
             o_ref, s_ref, c_ref):
        i = pl.program_id(0)

        @pl.when(i == 0)
        def _():
            s_ref[...] = jnp.zeros_like(s_ref)
            c_ref[...] = jnp.zeros_like(c_ref)

        s = acc_ref[0] + acc_ref[1] + g_ref[...]
        h = jnp.maximum(dinv_ref[...] * s + b_ref[...], 0.0)
        seg = lax.broadcasted_iota(jnp.int32, (NUM_GRAPHS, rb), 0)
        maskt = (seg == batch_ref[0]).astype(jnp.float32)
        s_ref[...] += jnp.dot(maskt, h, **_DOT)
        c_ref[...] = c_ref[...] + jnp.sum(maskt, axis=1, keepdims=True)

        @pl.when(i == nblk - 1)
        def _():
            pooled = s_ref[...] / jnp.maximum(c_ref[...], 1.0)
            o_ref[...] = jnp.dot(pooled, wc_ref[...], **_DOT) + bc_ref[...]

    return pl.pallas_call(
        body,
        grid=(nblk,),
        in_specs=[
            pl.BlockSpec((2, rb, d), lambda i: (0, i, 0)),
            pl.BlockSpec((rb, d), lambda i: (i, 0)),
            pl.BlockSpec((rb, 1), lambda i: (i, 0)),
            pl.BlockSpec((1, d), lambda i: (0, 0)),
            pl.BlockSpec((1, 1, rb), lambda i: (i, 0, 0)),
            pl.BlockSpec((d, dout), lambda i: (0, 0)),
            pl.BlockSpec((1, dout), lambda i: (0, 0)),
        ],
        out_specs=pl.BlockSpec((NUM_GRAPHS, dout), lambda i: (0, 0)),
        out_shape=jax.ShapeDtypeStruct((NUM_GRAPHS, dout), jnp.float32),
        scratch_shapes=[
            pltpu.VMEM((NUM_GRAPHS, d), jnp.float32),
            pltpu.VMEM((NUM_GRAPHS, d), jnp.float32),
        ],
    )(acc, g2, dinv_col, b2, batch3d, wc, bc)


def kernel(x, edge_index, batch, W1, b1, W2, b2, Wc, bc):
    n, d = x.shape
    src = edge_index[0].astype(jnp.int32)
    dst = edge_index[1].astype(jnp.int32)
    e = src.shape[0]

    n_acc = -(-n // 1024) * 1024
    n_chunks = -(-e // (NW * CHUNK * NBR)) * NBR
    e_pad = NW * n_chunks * CHUNK
    npad = e_pad - e
    pad_idx = jnp.arange(npad, dtype=jnp.int32)
    src_p = jnp.concatenate([src, pad_idx % n])
    dst_p = jnp.concatenate([dst, n + pad_idx % (n_acc - n)])

    degs = _sc_degree(dst_p, n_acc, n_chunks)
    h1 = _tc_matmul(x, W1)
    dinvp = _tc_degsum(degs, n_acc)
    dinv_col = dinvp.reshape(n_acc, 1)[:n]
    g1 = _tc_scale(h1, dinv_col)
    acc1 = _sc_edge_pass(g1, src_p, dst_p, n_acc, n_chunks)
    g2 = _tc_mid(acc1, g1, dinv_col, b1.reshape(1, -1), W2)
    acc2 = _sc_edge_pass(g2, src_p, dst_p, n_acc, n_chunks)
    return _tc_final(acc2, g2, dinv_col, b2.reshape(1, -1),
                     batch.astype(jnp.int32), Wc, bc.reshape(1, -1))


# speedup vs baseline: 33.9498x; 1.0067x over previous
import dataclasses
import functools

import jax
import jax.numpy as jnp
from jax import lax
from jax.experimental import pallas as pl
from jax.experimental.pallas import tpu as pltpu
from jax.experimental.pallas import tpu_sc as plsc

NC = 2
NS = 16
NW = NC * NS
CHUNK = 128
NB_D = 4
NB_E = 2
NBR = 4
NUM_GRAPHS = 64


def _mesh():
    return plsc.VectorSubcoreMesh(core_axis_name="c", subcore_axis_name="s")


def _sc_params():
    cp = pltpu.CompilerParams()
    if "needs_layout_passes" in pltpu.CompilerParams.__dataclass_fields__:
        cp = dataclasses.replace(cp, needs_layout_passes=False)
    return cp


def _fill(ref, rows, d, value):
    @pl.loop(0, rows)
    def _(r):
        @pl.loop(0, d, step=16)
        def _(c):
            ref[r, pl.ds(c, 16)] = jnp.full((16,), value, jnp.float32)


def _sc_degree(dst_p, n_acc, n_chunks):
    hr = n_acc // 128

    @functools.partial(
        pl.kernel,
        out_type=jax.ShapeDtypeStruct((NW, hr, 128), jnp.float32),
        mesh=_mesh(),
        scratch_types=[
            pltpu.VMEM((hr, 128), jnp.float32),
            pltpu.VMEM((4, CHUNK), jnp.int32),
            pltpu.SemaphoreType.DMA((4,)),
        ],
        compiler_params=_sc_params(),
    )
    def k(dst_hbm, out_hbm, hist, dstb, si):
        cid = lax.axis_index("c")
        sid = lax.axis_index("s")
        wid = cid * NS + sid
        _fill(hist, hr, 128, 0.0)
        base = wid * (n_chunks * CHUNK)

        def idx_start(s, ci):
            pltpu.async_copy(dst_hbm.at[pl.ds(base + ci * CHUNK, CHUNK)],
                             dstb.at[s], si.at[s])

        def idx_wait(s, ci):
            pltpu.make_async_copy(dst_hbm.at[pl.ds(base + ci * CHUNK, CHUNK)],
                                  dstb.at[s], si.at[s]).wait()

        for s in range(4):
            idx_start(s, s)

        @pl.loop(0, n_chunks // 4)
        def _(t):
            for j in range(4):
                ci = t * 4 + j
                idx_wait(j, ci)
                for g in range(CHUNK // 16):
                    v = dstb[j, pl.ds(g * 16, 16)]
                    cnt, last = plsc.scan_count(v)
                    plsc.addupdate_scatter(
                        hist,
                        [jax.lax.shift_right_logical(v, 7),
                         jax.lax.bitwise_and(v, 127)],
                        cnt.astype(jnp.float32), mask=last)

                @pl.when(ci + 4 < n_chunks)
                def _():
                    idx_start(j, ci + 4)

        pltpu.sync_copy(hist, out_hbm.at[wid])

    return k(dst_p)


def _sc_edge_pass(g, src_p, dst_p, n_acc, n_chunks):
    d = g.shape[1]

    @functools.partial(
        pl.kernel,
        out_type=jax.ShapeDtypeStruct((NC, n_acc, d), jnp.float32),
        mesh=_mesh(),
        scratch_types=[
            pltpu.VMEM_SHARED((n_acc, d), jnp.float32),
            pltpu.VMEM((4, CHUNK), jnp.int32),
            pltpu.VMEM((4, CHUNK), jnp.int32),
            pltpu.VMEM((2, CHUNK, d), jnp.float32),
            pltpu.VMEM((16, d), jnp.float32),
            pltpu.SemaphoreType.DMA((4,)),
            pltpu.SemaphoreType.DMA((2,)),
            pltpu.SemaphoreType.DMA((4,)),
        ],
    )
    def k(g_hbm, src_hbm, dst_hbm, out_hbm, acc, srcb, dstb, rows, zb,
          si, sg, ss):
        cid = lax.axis_index("c")
        sid = lax.axis_index("s")
        wid = cid * NS + sid
        _fill(zb, 16, d, 0.0)
        rps = n_acc // NS

        @pl.loop(0, rps // 16)
        def _(i):
            pltpu.sync_copy(zb, acc.at[pl.ds(sid * rps + i * 16, 16)])

        plsc.subcore_barrier()
        base = wid * (n_chunks * CHUNK)

        def idx_start(s, ci):
            off = base + ci * CHUNK
            pltpu.async_copy(src_hbm.at[pl.ds(off, CHUNK)], srcb.at[s], si.at[s])
            pltpu.async_copy(dst_hbm.at[pl.ds(off, CHUNK)], dstb.at[s], si.at[s])

        def idx_wait(s, ci):
            off = base + ci * CHUNK
            pltpu.make_async_copy(
                src_hbm.at[pl.ds(off, CHUNK)], srcb.at[s], si.at[s]).wait()
            pltpu.make_async_copy(
                dst_hbm.at[pl.ds(off, CHUNK)], dstb.at[s], si.at[s]).wait()

        def gather_start(r, s):
            pltpu.async_copy(g_hbm.at[srcb.at[s]], rows.at[r], sg.at[r])

        def gather_wait(r, s):
            pltpu.make_async_copy(g_hbm.at[srcb.at[s]], rows.at[r],
                                  sg.at[r]).wait()

        def scat_start(r, s):
            pltpu.async_copy(rows.at[r], acc.at[dstb.at[s]], ss.at[s], add=True)

        def scat_wait(r, s):
            pltpu.make_async_copy(rows.at[r], acc.at[dstb.at[s]],
                                  ss.at[s]).wait()

        for s in range(4):
            idx_start(s, s)

        @pl.loop(0, n_chunks // 4)
        def _(t):
            for j in range(4):
                ci = t * 4 + j
                sp = (j + 2) % 4
                sv = (j + 3) % 4

                def pre():
                    scat_wait(j % 2, sp)

                    @pl.when(ci + 2 < n_chunks)
                    def _():
                        idx_start(sp, ci + 2)

                if j >= 2:
                    pre()
                else:
                    @pl.when(t > 0)
                    def _():
                        pre()

                idx_wait(j, ci)
                gather_start(j % 2, j)

                def flow():
                    gather_wait((j + 1) % 2, sv)
                    scat_start((j + 1) % 2, sv)

                if j >= 1:
                    flow()
                else:
                    @pl.when(t > 0)
                    def _():
                        flow()

        gather_wait(1, 3)
        scat_start(1, 3)
        scat_wait(0, 2)
        scat_wait(1, 3)

        plsc.subcore_barrier()

        @pl.loop(0, rps // 64)
        def _(i):
            r0 = sid * rps + i * 64
            pltpu.sync_copy(acc.at[pl.ds(r0, 64)], out_hbm.at[cid, pl.ds(r0, 64)])

    return k(g, src_p, dst_p)


_DOT = dict(preferred_element_type=jnp.float32)


def _tc_in_proj(x, w, dinv_col):
    n, d = x.shape
    rb = 2000 if n % 2000 == 0 else n

    def body(x_ref, w_ref, dinv_ref, o_ref):
        o_ref[...] = jnp.dot(x_ref[...], w_ref[...], **_DOT) * dinv_ref[...]

    return pl.pallas_call(
        body,
        grid=(n // rb,),
        in_specs=[
            pl.BlockSpec((rb, d), lambda i: (i, 0)),
            pl.BlockSpec((d, w.shape[1]), lambda i: (0, 0)),
            pl.BlockSpec((rb, 1), lambda i: (i, 0)),
        ],
        out_specs=pl.BlockSpec((rb, w.shape[1]), lambda i: (i, 0)),
        out_shape=jax.ShapeDtypeStruct((n, w.shape[1]), jnp.float32),
    )(x, w, dinv_col)


def _tc_degsum(degs, n_acc):
    hr = n_acc // 128

    def body(degs_ref, o_ref):
        deg = jnp.sum(degs_ref[...], axis=0) + 1.0
        o_ref[...] = lax.rsqrt(jnp.maximum(deg, 1.0))

    return pl.pallas_call(
        body,
        out_shape=jax.ShapeDtypeStruct((hr, 128), jnp.float32),
    )(degs)


def _tc_mid(acc, g1, dinv_col, b1, w2):
    n, d = g1.shape
    rb = 2000 if n % 2000 == 0 else n

    def body(acc_ref, g_ref, dinv_ref, b_ref, w_ref, o_ref):
        s = acc_ref[0] + acc_ref[1] + g_ref[...]
        h = jnp.maximum(dinv_ref[...] * s + b_ref[...], 0.0)
        o_ref[...] = jnp.dot(h, w_ref[...], **_DOT) * dinv_ref[...]

    return pl.pallas_call(
        body,
        grid=(n // rb,),
        in_specs=[
            pl.BlockSpec((2, rb, d), lambda i: (0, i, 0)),
            pl.BlockSpec((rb, d), lambda i: (i, 0)),
            pl.BlockSpec((rb, 1), lambda i: (i, 0)),
            pl.BlockSpec((1, d), lambda i: (0, 0)),
            pl.BlockSpec((d, d), lambda i: (0, 0)),
        ],
        out_specs=pl.BlockSpec((rb, d), lambda i: (i, 0)),
        out_shape=jax.ShapeDtypeStruct((n, d), jnp.float32),
    )(acc, g1, dinv_col, b1, w2)


def _tc_final(acc, g2, dinv_col, b2, batch, wc, bc):
    n, d = g2.shape
    dout = wc.shape[1]
    rb = 2000 if n % 2000 == 0 else n
    nblk = n // rb
    batch3d = batch.reshape(nblk, 1, rb)

    def body(acc_ref, g_ref, dinv_ref, b_ref, batch_ref, wc_ref, bc_ref,
             o_ref, s_ref, c_ref):
        i = pl.program_id(0)

        @pl.when(i == 0)
        def _():
            s_ref[...] = jnp.zeros_like(s_ref)
            c_ref[...] = jnp.zeros_like(c_ref)

        s = acc_ref[0] + acc_ref[1] + g_ref[...]
        h = jnp.maximum(dinv_ref[...] * s + b_ref[...], 0.0)
        seg = lax.broadcasted_iota(jnp.int32, (NUM_GRAPHS, rb), 0)
        maskt = (seg == batch_ref[0]).astype(jnp.float32)
        s_ref[...] += jnp.dot(maskt, h, **_DOT)
        c_ref[...] = c_ref[...] + jnp.sum(maskt, axis=1, keepdims=True)

        @pl.when(i == nblk - 1)
        def _():
            pooled = s_ref[...] / jnp.maximum(c_ref[...], 1.0)
            o_ref[...] = jnp.dot(pooled, wc_ref[...], **_DOT) + bc_ref[...]

    return pl.pallas_call(
        body,
        grid=(nblk,),
        in_specs=[
            pl.BlockSpec((2, rb, d), lambda i: (0, i, 0)),
            pl.BlockSpec((rb, d), lambda i: (i, 0)),
            pl.BlockSpec((rb, 1), lambda i: (i, 0)),
            pl.BlockSpec((1, d), lambda i: (0, 0)),
            pl.BlockSpec((1, 1, rb), lambda i: (i, 0, 0)),
            pl.BlockSpec((d, dout), lambda i: (0, 0)),
            pl.BlockSpec((1, dout), lambda i: (0, 0)),
        ],
        out_specs=pl.BlockSpec((NUM_GRAPHS, dout), lambda i: (0, 0)),
        out_shape=jax.ShapeDtypeStruct((NUM_GRAPHS, dout), jnp.float32),
        scratch_shapes=[
            pltpu.VMEM((NUM_GRAPHS, d), jnp.float32),
            pltpu.VMEM((NUM_GRAPHS, d), jnp.float32),
        ],
    )(acc, g2, dinv_col, b2, batch3d, wc, bc)


def kernel(x, edge_index, batch, W1, b1, W2, b2, Wc, bc):
    n, d = x.shape
    src = edge_index[0].astype(jnp.int32)
    dst = edge_index[1].astype(jnp.int32)
    e = src.shape[0]

    n_acc = -(-n // 1024) * 1024
    n_chunks = -(-e // (NW * CHUNK * NBR)) * NBR
    e_pad = NW * n_chunks * CHUNK
    npad = e_pad - e
    pad_idx = jnp.arange(npad, dtype=jnp.int32)
    src_p = jnp.concatenate([src, pad_idx % n])
    dst_p = jnp.concatenate([dst, n + pad_idx % (n_acc - n)])

    degs = _sc_degree(dst_p, n_acc, n_chunks)
    dinvp = _tc_degsum(degs, n_acc)
    dinv_col = dinvp.reshape(n_acc, 1)[:n]
    g1 = _tc_in_proj(x, W1, dinv_col)
    acc1 = _sc_edge_pass(g1, src_p, dst_p, n_acc, n_chunks)
    g2 = _tc_mid(acc1, g1, dinv_col, b1.reshape(1, -1), W2)
    acc2 = _sc_edge_pass(g2, src_p, dst_p, n_acc, n_chunks)
    return _tc_final(acc2, g2, dinv_col, b2.reshape(1, -1),
                     batch.astype(jnp.int32), Wc, bc.reshape(1, -1))


# speedup vs baseline: 34.0187x; 1.0020x over previous
import dataclasses
import functools

import jax
import jax.numpy as jnp
from jax import lax
from jax.experimental import pallas as pl
from jax.experimental.pallas import tpu as pltpu
from jax.experimental.pallas import tpu_sc as plsc

NC = 2
NS = 16
NW = NC * NS
CHUNK = 128
NBR = 4
NUM_GRAPHS = 64


def _mesh():
    return plsc.VectorSubcoreMesh(core_axis_name="c", subcore_axis_name="s")


def _sc_params():
    cp = pltpu.CompilerParams()
    if "needs_layout_passes" in pltpu.CompilerParams.__dataclass_fields__:
        cp = dataclasses.replace(cp, needs_layout_passes=False)
    return cp


def _fill(ref, rows, d, value):
    @pl.loop(0, rows)
    def _(r):
        @pl.loop(0, d, step=16)
        def _(c):
            ref[r, pl.ds(c, 16)] = jnp.full((16,), value, jnp.float32)


def _sc_degree(dst_p, n_acc, n_chunks):
    hr = n_acc // 128

    @functools.partial(
        pl.kernel,
        out_type=jax.ShapeDtypeStruct((NW, hr, 128), jnp.float32),
        mesh=_mesh(),
        scratch_types=[
            pltpu.VMEM((hr, 128), jnp.float32),
            pltpu.VMEM((4, CHUNK), jnp.int32),
            pltpu.SemaphoreType.DMA((4,)),
        ],
        compiler_params=_sc_params(),
    )
    def k(dst_hbm, out_hbm, hist, dstb, si):
        cid = lax.axis_index("c")
        sid = lax.axis_index("s")
        wid = cid * NS + sid
        _fill(hist, hr, 128, 0.0)
        base = wid * (n_chunks * CHUNK)

        def idx_start(s, ci):
            pltpu.async_copy(dst_hbm.at[pl.ds(base + ci * CHUNK, CHUNK)],
                             dstb.at[s], si.at[s])

        def idx_wait(s, ci):
            pltpu.make_async_copy(dst_hbm.at[pl.ds(base + ci * CHUNK, CHUNK)],
                                  dstb.at[s], si.at[s]).wait()

        for s in range(4):
            idx_start(s, s)

        @pl.loop(0, n_chunks // 4)
        def _(t):
            for j in range(4):
                ci = t * 4 + j
                idx_wait(j, ci)
                for g in range(CHUNK // 16):
                    v = dstb[j, pl.ds(g * 16, 16)]
                    cnt, last = plsc.scan_count(v)
                    plsc.addupdate_scatter(
                        hist,
                        [jax.lax.shift_right_logical(v, 7),
                         jax.lax.bitwise_and(v, 127)],
                        cnt.astype(jnp.float32), mask=last)

                @pl.when(ci + 4 < n_chunks)
                def _():
                    idx_start(j, ci + 4)

        pltpu.sync_copy(hist, out_hbm.at[wid])

    return k(dst_p)


def _sc_edge_pass(g, src_p, dst_p, n_acc, n_chunks):
    d = g.shape[1]

    @functools.partial(
        pl.kernel,
        out_type=jax.ShapeDtypeStruct((NC, n_acc, d), jnp.float32),
        mesh=_mesh(),
        scratch_types=[
            pltpu.VMEM_SHARED((n_acc, d), jnp.float32),
            pltpu.VMEM((4, CHUNK), jnp.int32),
            pltpu.VMEM((4, CHUNK), jnp.int32),
            pltpu.VMEM((2, CHUNK, d), jnp.float32),
            pltpu.VMEM((16, d), jnp.float32),
            pltpu.SemaphoreType.DMA((4,)),
            pltpu.SemaphoreType.DMA((2,)),
            pltpu.SemaphoreType.DMA((4,)),
        ],
    )
    def k(g_hbm, src_hbm, dst_hbm, out_hbm, acc, srcb, dstb, rows, zb,
          si, sg, ss):
        cid = lax.axis_index("c")
        sid = lax.axis_index("s")
        wid = cid * NS + sid
        _fill(zb, 16, d, 0.0)
        rps = n_acc // NS

        @pl.loop(0, rps // 16)
        def _(i):
            pltpu.sync_copy(zb, acc.at[pl.ds(sid * rps + i * 16, 16)])

        plsc.subcore_barrier()
        base = wid * (n_chunks * CHUNK)

        def idx_start(s, ci):
            off = base + ci * CHUNK
            pltpu.async_copy(src_hbm.at[pl.ds(off, CHUNK)], srcb.at[s], si.at[s])
            pltpu.async_copy(dst_hbm.at[pl.ds(off, CHUNK)], dstb.at[s], si.at[s])

        def idx_wait(s, ci):
            off = base + ci * CHUNK
            pltpu.make_async_copy(
                src_hbm.at[pl.ds(off, CHUNK)], srcb.at[s], si.at[s]).wait()
            pltpu.make_async_copy(
                dst_hbm.at[pl.ds(off, CHUNK)], dstb.at[s], si.at[s]).wait()

        def gather_start(r, s):
            pltpu.async_copy(g_hbm.at[srcb.at[s]], rows.at[r], sg.at[r])

        def gather_wait(r, s):
            pltpu.make_async_copy(g_hbm.at[srcb.at[s]], rows.at[r],
                                  sg.at[r]).wait()

        def scat_start(r, s):
            pltpu.async_copy(rows.at[r], acc.at[dstb.at[s]], ss.at[s], add=True)

        def scat_wait(r, s):
            pltpu.make_async_copy(rows.at[r], acc.at[dstb.at[s]],
                                  ss.at[s]).wait()

        for s in range(4):
            idx_start(s, s)

        @pl.loop(0, n_chunks // 4)
        def _(t):
            for j in range(4):
                ci = t * 4 + j
                sp = (j + 2) % 4
                sv = (j + 3) % 4

                def pre():
                    scat_wait(j % 2, sp)

                    @pl.when(ci + 2 < n_chunks)
                    def _():
                        idx_start(sp, ci + 2)

                if j >= 2:
                    pre()
                else:
                    @pl.when(t > 0)
                    def _():
                        pre()

                idx_wait(j, ci)
                gather_start(j % 2, j)

                def flow():
                    gather_wait((j + 1) % 2, sv)
                    scat_start((j + 1) % 2, sv)

                if j >= 1:
                    flow()
                else:
                    @pl.when(t > 0)
                    def _():
                        flow()

        gather_wait(1, 3)
        scat_start(1, 3)
        scat_wait(0, 2)
        scat_wait(1, 3)

        plsc.subcore_barrier()

        @pl.loop(0, rps // 64)
        def _(i):
            r0 = sid * rps + i * 64
            pltpu.sync_copy(acc.at[pl.ds(r0, 64)], out_hbm.at[cid, pl.ds(r0, 64)])

    return k(g, src_p, dst_p)


_DOT = dict(preferred_element_type=jnp.float32)


def _tc_in_proj(x, w, dinv_col):
    n, d = x.shape
    rb = 2000 if n % 2000 == 0 else n

    def body(x_ref, w_ref, dinv_ref, o_ref):
        o_ref[...] = jnp.dot(x_ref[...], w_ref[...], **_DOT) * dinv_ref[...]

    return pl.pallas_call(
        body,
        grid=(n // rb,),
        in_specs=[
            pl.BlockSpec((rb, d), lambda i: (i, 0)),
            pl.BlockSpec((d, w.shape[1]), lambda i: (0, 0)),
            pl.BlockSpec((rb, 1), lambda i: (i, 0)),
        ],
        out_specs=pl.BlockSpec((rb, w.shape[1]), lambda i: (i, 0)),
        out_shape=jax.ShapeDtypeStruct((n, w.shape[1]), jnp.float32),
    )(x, w, dinv_col)


def _tc_degsum(degs, n_acc):
    hr = n_acc // 128

    def body(degs_ref, o_ref):
        deg = jnp.sum(degs_ref[...], axis=0) + 1.0
        o_ref[...] = lax.rsqrt(jnp.maximum(deg, 1.0))

    return pl.pallas_call(
        body,
        out_shape=jax.ShapeDtypeStruct((hr, 128), jnp.float32),
    )(degs)


def _tc_mid(acc, g1, dinv_col, b1, w2):
    n, d = g1.shape
    rb = 2000 if n % 2000 == 0 else n

    def body(acc_ref, g_ref, dinv_ref, b_ref, w_ref, o_ref):
        s = acc_ref[0] + acc_ref[1] + g_ref[...]
        h = jnp.maximum(dinv_ref[...] * s + b_ref[...], 0.0)
        o_ref[...] = jnp.dot(h, w_ref[...], **_DOT) * dinv_ref[...]

    return pl.pallas_call(
        body,
        grid=(n // rb,),
        in_specs=[
            pl.BlockSpec((2, rb, d), lambda i: (0, i, 0)),
            pl.BlockSpec((rb, d), lambda i: (i, 0)),
            pl.BlockSpec((rb, 1), lambda i: (i, 0)),
            pl.BlockSpec((1, d), lambda i: (0, 0)),
            pl.BlockSpec((d, d), lambda i: (0, 0)),
        ],
        out_specs=pl.BlockSpec((rb, d), lambda i: (i, 0)),
        out_shape=jax.ShapeDtypeStruct((n, d), jnp.float32),
    )(acc, g1, dinv_col, b1, w2)


def _tc_final(acc, g2, dinv_col, b2, batch, wc, bc):
    n, d = g2.shape
    dout = wc.shape[1]
    rb = 2000 if n % 2000 == 0 else n
    nblk = n // rb
    batch3d = batch.reshape(nblk, 1, rb)

    def body(acc_ref, g_ref, dinv_ref, b_ref, batch_ref, wc_ref, bc_ref,
             o_ref, s_ref, c_ref):
        i = pl.program_id(0)

        @pl.when(i == 0)
        def _():
            s_ref[...] = jnp.zeros_like(s_ref)
            c_ref[...] = jnp.zeros_like(c_ref)

        s = acc_ref[0] + acc_ref[1] + g_ref[...]
        h = jnp.maximum(dinv_ref[...] * s + b_ref[...], 0.0)
        seg = lax.broadcasted_iota(jnp.int32, (NUM_GRAPHS, rb), 0)
        maskt = (seg == batch_ref[0]).astype(jnp.float32)
        s_ref[...] += jnp.dot(maskt, h, **_DOT)
        c_ref[...] = c_ref[...] + jnp.sum(maskt, axis=1, keepdims=True)

        @pl.when(i == nblk - 1)
        def _():
            pooled = s_ref[...] / jnp.maximum(c_ref[...], 1.0)
            o_ref[...] = jnp.dot(pooled, wc_ref[...], **_DOT) + bc_ref[...]

    return pl.pallas_call(
        body,
        grid=(nblk,),
        in_specs=[
            pl.BlockSpec((2, rb, d), lambda i: (0, i, 0)),
            pl.BlockSpec((rb, d), lambda i: (i, 0)),
            pl.BlockSpec((rb, 1), lambda i: (i, 0)),
            pl.BlockSpec((1, d), lambda i: (0, 0)),
            pl.BlockSpec((1, 1, rb), lambda i: (i, 0, 0)),
            pl.BlockSpec((d, dout), lambda i: (0, 0)),
            pl.BlockSpec((1, dout), lambda i: (0, 0)),
        ],
        out_specs=pl.BlockSpec((NUM_GRAPHS, dout), lambda i: (0, 0)),
        out_shape=jax.ShapeDtypeStruct((NUM_GRAPHS, dout), jnp.float32),
        scratch_shapes=[
            pltpu.VMEM((NUM_GRAPHS, d), jnp.float32),
            pltpu.VMEM((NUM_GRAPHS, d), jnp.float32),
        ],
    )(acc, g2, dinv_col, b2, batch3d, wc, bc)


def kernel(x, edge_index, batch, W1, b1, W2, b2, Wc, bc):
    n, d = x.shape
    src = edge_index[0].astype(jnp.int32)
    dst = edge_index[1].astype(jnp.int32)
    e = src.shape[0]

    n_acc = -(-n // 1024) * 1024
    n_chunks = -(-e // (NW * CHUNK * NBR)) * NBR
    e_pad = NW * n_chunks * CHUNK
    npad = e_pad - e
    pad_idx = jnp.arange(npad, dtype=jnp.int32)
    src_p = jnp.concatenate([src, pad_idx % n])
    dst_p = jnp.concatenate([dst, n + pad_idx % (n_acc - n)])

    degs = _sc_degree(dst_p, n_acc, n_chunks)
    dinvp = _tc_degsum(degs, n_acc)
    dinv_col = dinvp.reshape(n_acc, 1)[:n]
    g1 = _tc_in_proj(x, W1, dinv_col)
    acc1 = _sc_edge_pass(g1, src_p, dst_p, n_acc, n_chunks)
    g2 = _tc_mid(acc1, g1, dinv_col, b1.reshape(1, -1), W2)
    acc2 = _sc_edge_pass(g2, src_p, dst_p, n_acc, n_chunks)
    return _tc_final(acc2, g2, dinv_col, b2.reshape(1, -1),
                     batch.astype(jnp.int32), Wc, bc.reshape(1, -1))


# speedup vs baseline: 34.5307x; 1.0151x over previous
import dataclasses
import functools

import jax
import jax.numpy as jnp
from jax import lax
from jax.experimental import pallas as pl
from jax.experimental.pallas import tpu as pltpu
from jax.experimental.pallas import tpu_sc as plsc

NC = 2
NS = 16
NW = NC * NS
CHUNK = 128
NBR = 4
NUM_GRAPHS = 64


def _mesh():
    return plsc.VectorSubcoreMesh(core_axis_name="c", subcore_axis_name="s")


def _sc_params():
    cp = pltpu.CompilerParams()
    if "needs_layout_passes" in pltpu.CompilerParams.__dataclass_fields__:
        cp = dataclasses.replace(cp, needs_layout_passes=False)
    return cp


def _fill(ref, rows, d, value):
    @pl.loop(0, rows)
    def _(r):
        @pl.loop(0, d, step=16)
        def _(c):
            ref[r, pl.ds(c, 16)] = jnp.full((16,), value, jnp.float32)


def _sc_degree(dst_p, n_acc, n_chunks):
    hr = n_acc // 128

    @functools.partial(
        pl.kernel,
        out_type=jax.ShapeDtypeStruct((NW, hr, 128), jnp.float32),
        mesh=_mesh(),
        scratch_types=[
            pltpu.VMEM((hr, 128), jnp.float32),
            pltpu.VMEM((4, CHUNK), jnp.int32),
            pltpu.SemaphoreType.DMA((4,)),
        ],
        compiler_params=_sc_params(),
    )
    def k(dst_hbm, out_hbm, hist, dstb, si):
        cid = lax.axis_index("c")
        sid = lax.axis_index("s")
        wid = cid * NS + sid
        _fill(hist, hr, 128, 0.0)
        base = wid * (n_chunks * CHUNK)

        def idx_start(s, ci):
            pltpu.async_copy(dst_hbm.at[pl.ds(base + ci * CHUNK, CHUNK)],
                             dstb.at[s], si.at[s])

        def idx_wait(s, ci):
            pltpu.make_async_copy(dst_hbm.at[pl.ds(base + ci * CHUNK, CHUNK)],
                                  dstb.at[s], si.at[s]).wait()

        for s in range(4):
            idx_start(s, s)

        @pl.loop(0, n_chunks // 4)
        def _(t):
            for j in range(4):
                ci = t * 4 + j
                idx_wait(j, ci)
                for g in range(CHUNK // 16):
                    v = dstb[j, pl.ds(g * 16, 16)]
                    cnt, last = plsc.scan_count(v)
                    plsc.addupdate_scatter(
                        hist,
                        [jax.lax.shift_right_logical(v, 7),
                         jax.lax.bitwise_and(v, 127)],
                        cnt.astype(jnp.float32), mask=last)

                @pl.when(ci + 4 < n_chunks)
                def _():
                    idx_start(j, ci + 4)

        pltpu.sync_copy(hist, out_hbm.at[wid])

    return k(dst_p)


def _sc_edge_pass(g, src_p, dst_p, n_acc, n_chunks):
    d = g.shape[1]

    @functools.partial(
        pl.kernel,
        out_type=jax.ShapeDtypeStruct((NC, n_acc, d), jnp.float32),
        mesh=_mesh(),
        scratch_types=[
            pltpu.VMEM_SHARED((n_acc, d), jnp.float32),
            pltpu.VMEM((4, CHUNK), jnp.int32),
            pltpu.VMEM((4, CHUNK), jnp.int32),
            pltpu.VMEM((2, CHUNK, d), jnp.float32),
            pltpu.VMEM((64, d), jnp.float32),
            pltpu.SemaphoreType.DMA((4,)),
            pltpu.SemaphoreType.DMA((2,)),
            pltpu.SemaphoreType.DMA((4,)),
        ],
    )
    def k(g_hbm, src_hbm, dst_hbm, out_hbm, acc, srcb, dstb, rows, zb,
          si, sg, ss):
        cid = lax.axis_index("c")
        sid = lax.axis_index("s")
        wid = cid * NS + sid
        _fill(zb, 64, d, 0.0)
        rps = n_acc // NS

        @pl.loop(0, rps // 64)
        def _(i):
            pltpu.async_copy(zb, acc.at[pl.ds(sid * rps + i * 64, 64)],
                             si.at[0])

        @pl.loop(0, rps // 64)
        def _(i):
            pltpu.make_async_copy(zb, acc.at[pl.ds(sid * rps + i * 64, 64)],
                                  si.at[0]).wait()

        plsc.subcore_barrier()
        base = wid * (n_chunks * CHUNK)

        def idx_start(s, ci):
            off = base + ci * CHUNK
            pltpu.async_copy(src_hbm.at[pl.ds(off, CHUNK)], srcb.at[s], si.at[s])
            pltpu.async_copy(dst_hbm.at[pl.ds(off, CHUNK)], dstb.at[s], si.at[s])

        def idx_wait(s, ci):
            off = base + ci * CHUNK
            pltpu.make_async_copy(
                src_hbm.at[pl.ds(off, CHUNK)], srcb.at[s], si.at[s]).wait()
            pltpu.make_async_copy(
                dst_hbm.at[pl.ds(off, CHUNK)], dstb.at[s], si.at[s]).wait()

        def gather_start(r, s):
            pltpu.async_copy(g_hbm.at[srcb.at[s]], rows.at[r], sg.at[r])

        def gather_wait(r, s):
            pltpu.make_async_copy(g_hbm.at[srcb.at[s]], rows.at[r],
                                  sg.at[r]).wait()

        def scat_start(r, s):
            pltpu.async_copy(rows.at[r], acc.at[dstb.at[s]], ss.at[s], add=True)

        def scat_wait(r, s):
            pltpu.make_async_copy(rows.at[r], acc.at[dstb.at[s]],
                                  ss.at[s]).wait()

        for s in range(4):
            idx_start(s, s)

        @pl.loop(0, n_chunks // 4)
        def _(t):
            for j in range(4):
                ci = t * 4 + j
                sp = (j + 2) % 4
                sv = (j + 3) % 4

                def pre():
                    scat_wait(j % 2, sp)

                    @pl.when(ci + 2 < n_chunks)
                    def _():
                        idx_start(sp, ci + 2)

                if j >= 2:
                    pre()
                else:
                    @pl.when(t > 0)
                    def _():
                        pre()

                idx_wait(j, ci)
                gather_start(j % 2, j)

                def flow():
                    gather_wait((j + 1) % 2, sv)
                    scat_start((j + 1) % 2, sv)

                if j >= 1:
                    flow()
                else:
                    @pl.when(t > 0)
                    def _():
                        flow()

        gather_wait(1, 3)
        scat_start(1, 3)
        scat_wait(0, 2)
        scat_wait(1, 3)

        plsc.subcore_barrier()

        @pl.loop(0, rps // 64)
        def _(i):
            r0 = sid * rps + i * 64
            pltpu.async_copy(acc.at[pl.ds(r0, 64)], out_hbm.at[cid, pl.ds(r0, 64)],
                             si.at[1])

        @pl.loop(0, rps // 64)
        def _(i):
            r0 = sid * rps + i * 64
            pltpu.make_async_copy(acc.at[pl.ds(r0, 64)],
                                  out_hbm.at[cid, pl.ds(r0, 64)], si.at[1]).wait()

    return k(g, src_p, dst_p)


_DOT = dict(preferred_element_type=jnp.float32)


def _tc_in_proj(x, w, dinv_col):
    n, d = x.shape
    rb = 2000 if n % 2000 == 0 else n

    def body(x_ref, w_ref, dinv_ref, o_ref):
        o_ref[...] = jnp.dot(x_ref[...], w_ref[...], **_DOT) * dinv_ref[...]

    return pl.pallas_call(
        body,
        grid=(n // rb,),
        in_specs=[
            pl.BlockSpec((rb, d), lambda i: (i, 0)),
            pl.BlockSpec((d, w.shape[1]), lambda i: (0, 0)),
            pl.BlockSpec((rb, 1), lambda i: (i, 0)),
        ],
        out_specs=pl.BlockSpec((rb, w.shape[1]), lambda i: (i, 0)),
        out_shape=jax.ShapeDtypeStruct((n, w.shape[1]), jnp.float32),
    )(x, w, dinv_col)


def _tc_degsum(degs, n_acc):
    hr = n_acc // 128

    def body(degs_ref, o_ref):
        deg = jnp.sum(degs_ref[...], axis=0) + 1.0
        o_ref[...] = lax.rsqrt(jnp.maximum(deg, 1.0))

    return pl.pallas_call(
        body,
        out_shape=jax.ShapeDtypeStruct((hr, 128), jnp.float32),
    )(degs)


def _tc_mid(acc, g1, dinv_col, b1, w2):
    n, d = g1.shape
    rb = 2000 if n % 2000 == 0 else n

    def body(acc_ref, g_ref, dinv_ref, b_ref, w_ref, o_ref):
        s = acc_ref[0] + acc_ref[1] + g_ref[...]
        h = jnp.maximum(dinv_ref[...] * s + b_ref[...], 0.0)
        o_ref[...] = jnp.dot(h, w_ref[...], **_DOT) * dinv_ref[...]

    return pl.pallas_call(
        body,
        grid=(n // rb,),
        in_specs=[
            pl.BlockSpec((2, rb, d), lambda i: (0, i, 0)),
            pl.BlockSpec((rb, d), lambda i: (i, 0)),
            pl.BlockSpec((rb, 1), lambda i: (i, 0)),
            pl.BlockSpec((1, d), lambda i: (0, 0)),
            pl.BlockSpec((d, d), lambda i: (0, 0)),
        ],
        out_specs=pl.BlockSpec((rb, d), lambda i: (i, 0)),
        out_shape=jax.ShapeDtypeStruct((n, d), jnp.float32),
    )(acc, g1, dinv_col, b1, w2)


def _tc_final(acc, g2, dinv_col, b2, batch, wc, bc):
    n, d = g2.shape
    dout = wc.shape[1]
    rb = 2000 if n % 2000 == 0 else n
    nblk = n // rb
    batch3d = batch.reshape(nblk, 1, rb)

    def body(acc_ref, g_ref, dinv_ref, b_ref, batch_ref, wc_ref, bc_ref,
             o_ref, s_ref, c_ref):
        i = pl.program_id(0)

        @pl.when(i == 0)
        def _():
            s_ref[...] = jnp.zeros_like(s_ref)
            c_ref[...] = jnp.zeros_like(c_ref)

        s = acc_ref[0] + acc_ref[1] + g_ref[...]
        h = jnp.maximum(dinv_ref[...] * s + b_ref[...], 0.0)
        seg = lax.broadcasted_iota(jnp.int32, (NUM_GRAPHS, rb), 0)
        maskt = (seg == batch_ref[0]).astype(jnp.float32)
        s_ref[...] += jnp.dot(maskt, h, **_DOT)
        c_ref[...] = c_ref[...] + jnp.sum(maskt, axis=1, keepdims=True)

        @pl.when(i == nblk - 1)
        def _():
            pooled = s_ref[...] / jnp.maximum(c_ref[...], 1.0)
            o_ref[...] = jnp.dot(pooled, wc_ref[...], **_DOT) + bc_ref[...]

    return pl.pallas_call(
        body,
        grid=(nblk,),
        in_specs=[
            pl.BlockSpec((2, rb, d), lambda i: (0, i, 0)),
            pl.BlockSpec((rb, d), lambda i: (i, 0)),
            pl.BlockSpec((rb, 1), lambda i: (i, 0)),
            pl.BlockSpec((1, d), lambda i: (0, 0)),
            pl.BlockSpec((1, 1, rb), lambda i: (i, 0, 0)),
            pl.BlockSpec((d, dout), lambda i: (0, 0)),
            pl.BlockSpec((1, dout), lambda i: (0, 0)),
        ],
        out_specs=pl.BlockSpec((NUM_GRAPHS, dout), lambda i: (0, 0)),
        out_shape=jax.ShapeDtypeStruct((NUM_GRAPHS, dout), jnp.float32),
        scratch_shapes=[
            pltpu.VMEM((NUM_GRAPHS, d), jnp.float32),
            pltpu.VMEM((NUM_GRAPHS, d), jnp.float32),
        ],
    )(acc, g2, dinv_col, b2, batch3d, wc, bc)


def kernel(x, edge_index, batch, W1, b1, W2, b2, Wc, bc):
    n, d = x.shape
    src = edge_index[0].astype(jnp.int32)
    dst = edge_index[1].astype(jnp.int32)
    e = src.shape[0]

    n_acc = -(-n // 1024) * 1024
    n_chunks = -(-e // (NW * CHUNK * NBR)) * NBR
    e_pad = NW * n_chunks * CHUNK
    npad = e_pad - e
    pad_idx = jnp.arange(npad, dtype=jnp.int32)
    src_p = jnp.concatenate([src, pad_idx % n])
    dst_p = jnp.concatenate([dst, n + pad_idx % (n_acc - n)])

    degs = _sc_degree(dst_p, n_acc, n_chunks)
    dinvp = _tc_degsum(degs, n_acc)
    dinv_col = dinvp.reshape(n_acc, 1)[:n]
    g1 = _tc_in_proj(x, W1, dinv_col)
    acc1 = _sc_edge_pass(g1, src_p, dst_p, n_acc, n_chunks)
    g2 = _tc_mid(acc1, g1, dinv_col, b1.reshape(1, -1), W2)
    acc2 = _sc_edge_pass(g2, src_p, dst_p, n_acc, n_chunks)
    return _tc_final(acc2, g2, dinv_col, b2.reshape(1, -1),
                     batch.astype(jnp.int32), Wc, bc.reshape(1, -1))
